# Initial kernel scaffold; baseline (speedup 1.0000x reference)
import jax
import jax.numpy as jnp
from jax.experimental import pallas as pl


def kernel(input_ids, token_emb, pos_emb, qW, qb, kW, kb, vW, vb, sW, sb, recipe, WdW, Wdb, n1g, n1b, n2g, n2b, ng, nb, basis_A, basis_B, basis_emb):
    raise NotImplementedError("write your pallas kernel here")


# speedup vs baseline: 1.2452x; 1.2452x over previous
import functools

import jax
import jax.numpy as jnp
from jax import lax
from jax.experimental import pallas as pl
from jax.experimental.pallas import tpu as pltpu
from jax.experimental.pallas import tpu_sc as plsc

S, D, H, DH = 2048, 768, 12, 64
NB, R, DFF, NN, K = 32, 64, 1024, 64, 8
L = 4
BS = 256
ASB = 512
VB = 2048
EPS = 1e-5


def _lnorm(x, g, b):
    mu = jnp.mean(x, axis=-1, keepdims=True)
    var = jnp.mean((x - mu) ** 2, axis=-1, keepdims=True)
    return (x - mu) * lax.rsqrt(var + EPS) * g + b


def _embed_gather(table, ids):
    info = plsc.get_sparse_core_info()
    nw = info.num_cores * info.num_subcores
    n = ids.shape[0]
    bpw = n // nw
    d = table.shape[1]
    mesh = plsc.VectorSubcoreMesh(core_axis_name="c", subcore_axis_name="s")

    @functools.partial(
        pl.kernel, mesh=mesh,
        out_type=jax.ShapeDtypeStruct((n, d), table.dtype),
        scratch_types=[
            pltpu.VMEM((bpw,), jnp.int32),
            pltpu.VMEM((bpw, d), table.dtype),
            pltpu.SemaphoreType.DMA,
        ],
    )
    def gather_k(table_hbm, idx_hbm, out_hbm, idx_v, rows_v, sem):
        wid = lax.axis_index("s") * info.num_cores + lax.axis_index("c")
        base = wid * bpw
        pltpu.sync_copy(idx_hbm.at[pl.ds(base, bpw)], idx_v)
        pltpu.async_copy(table_hbm.at[idx_v], rows_v, sem).wait()
        pltpu.sync_copy(rows_v, out_hbm.at[pl.ds(base, bpw)])

    return gather_k(table, ids)


def _embed_finish_body(g_ref, pos_ref, g1_ref, b1_ref, x_ref, h1_ref):
    x = g_ref[...] + pos_ref[...]
    x_ref[...] = x
    h1_ref[...] = _lnorm(x, g1_ref[...], b1_ref[...])


def _embed_finish(g, pos, g1, b1):
    blk = pl.BlockSpec((BS, D), lambda i: (i, 0))
    vec = pl.BlockSpec((1, D), lambda i: (0, 0))
    return pl.pallas_call(
        _embed_finish_body,
        grid=(S // BS,),
        in_specs=[blk, blk, vec, vec],
        out_specs=[pl.BlockSpec((BS, D), lambda i: (i, 0))] * 2,
        out_shape=[jax.ShapeDtypeStruct((S, D), jnp.float32)] * 2,
    )(g, pos, g1.reshape(1, D), b1.reshape(1, D))


def _attn_body(h1_ref, qW_ref, kW_ref, vW_ref, qb_ref, kb_ref, vb_ref,
               o_ref, kk_s, vv_s):
    sblk = pl.program_id(1)

    @pl.when(sblk == 0)
    def _():
        h1 = h1_ref[...]
        kk_s[...] = (jnp.dot(h1, kW_ref[0], preferred_element_type=jnp.float32)
                     + kb_ref[0])
        vv_s[...] = (jnp.dot(h1, vW_ref[0], preferred_element_type=jnp.float32)
                     + vb_ref[0])

    q = (jnp.dot(h1_ref[pl.ds(sblk * ASB, ASB), :], qW_ref[0],
                 preferred_element_type=jnp.float32) + qb_ref[0])
    scores = lax.dot_general(q, kk_s[...], (((1,), (1,)), ((), ())),
                             preferred_element_type=jnp.float32)
    scores = scores * (1.0 / (DH ** 0.5))
    m = jnp.max(scores, axis=-1, keepdims=True)
    p = jnp.exp(scores - m)
    denom = jnp.sum(p, axis=-1, keepdims=True)
    ctx = jnp.dot(p, vv_s[...], preferred_element_type=jnp.float32) / denom
    o_ref[0] = ctx


def _attn(h1, qWh, kWh, vWh, qbh, kbh, vbh):
    wspec = pl.BlockSpec((1, D, DH), lambda h, s: (h, 0, 0))
    bspec = pl.BlockSpec((1, 1, DH), lambda h, s: (h, 0, 0))
    return pl.pallas_call(
        _attn_body,
        grid=(H, S // ASB),
        in_specs=[pl.BlockSpec((S, D), lambda h, s: (0, 0)),
                  wspec, wspec, wspec, bspec, bspec, bspec],
        out_specs=pl.BlockSpec((1, ASB, DH), lambda h, s: (h, s, 0)),
        out_shape=jax.ShapeDtypeStruct((H, S, DH), jnp.float32),
        scratch_shapes=[pltpu.VMEM((S, DH), jnp.float32),
                        pltpu.VMEM((S, DH), jnp.float32)],
        compiler_params=pltpu.CompilerParams(
            dimension_semantics=("arbitrary", "arbitrary")),
    )(h1, qWh, kWh, vWh, qbh, kbh, vbh)


def _ffn_body(x_ref, h1_ref, ctx_ref, sW1_ref, sW2_ref, sb_ref, rec_ref,
              bemb_ref, Af_ref, Bf_ref, WdW_ref, Wdb_ref, n2g_ref, n2b_ref,
              gn_ref, bn_ref, rep_ref, summ_ref, tile_ref, xo_ref, h1o_ref):
    h1 = h1_ref[...]
    ctx = jnp.concatenate([ctx_ref[h] for h in range(H)], axis=-1)
    query = (jnp.dot(h1, sW1_ref[...], preferred_element_type=jnp.float32)
             + jnp.dot(ctx, sW2_ref[...], preferred_element_type=jnp.float32)
             + sb_ref[...])

    rec = rec_ref[...]
    er = jnp.exp(rec - jnp.max(rec, axis=-1, keepdims=True))
    srec = er / jnp.sum(er, axis=-1, keepdims=True)
    nemb = jnp.dot(srec, bemb_ref[...], preferred_element_type=jnp.float32)
    scores = lax.dot_general(query, nemb, (((1,), (1,)), ((), ())),
                             preferred_element_type=jnp.float32)

    iota = lax.broadcasted_iota(jnp.int32, (BS, NN), 1)
    work = scores
    selmask = jnp.zeros((BS, NN), jnp.bool_)
    for _ in range(K):
        cm = jnp.max(work, axis=-1, keepdims=True)
        cand = jnp.where(work == cm, iota, NN)
        first = jnp.min(cand, axis=-1, keepdims=True)
        onehot = iota == first
        selmask = jnp.logical_or(selmask, onehot)
        work = jnp.where(onehot, -jnp.inf, work)
    gmax = jnp.max(scores, axis=-1, keepdims=True)
    wnum = jnp.where(selmask, jnp.exp(scores - gmax), 0.0)
    wd = wnum / jnp.sum(wnum, axis=-1, keepdims=True)
    tr = jnp.dot(wd, srec, preferred_element_type=jnp.float32)

    x = x_ref[...]
    h2 = _lnorm(x, n2g_ref[...], n2b_ref[...])
    u = jnp.dot(h2, Af_ref[...], preferred_element_type=jnp.float32)
    t = jnp.dot(tr, rep_ref[...], preferred_element_type=jnp.float32)
    hh = jnp.dot(u * t, summ_ref[...], preferred_element_type=jnp.float32)
    hrep = jnp.dot(hh, tile_ref[...], preferred_element_type=jnp.float32)
    ff = jnp.dot(t * hrep, Bf_ref[...], preferred_element_type=jnp.float32)
    ff = ff * 0.5 * (1.0 + lax.erf(ff * (2.0 ** -0.5)))
    y = jnp.dot(ff, WdW_ref[...], preferred_element_type=jnp.float32) + Wdb_ref[...]
    xn = x + y
    xo_ref[...] = xn
    h1o_ref[...] = _lnorm(xn, gn_ref[...], bn_ref[...])


def _ffn(x, h1, ctx, sW1, sW2, sb, rec, bemb, Af, Bf, WdWl, Wdbl,
         n2gl, n2bl, gn, bn, rep, summ, tile):
    blk = pl.BlockSpec((BS, D), lambda i: (i, 0))
    vec = pl.BlockSpec((1, D), lambda i: (0, 0))
    full = lambda shape: pl.BlockSpec(shape, lambda i: (0,) * len(shape))
    return pl.pallas_call(
        _ffn_body,
        grid=(S // BS,),
        in_specs=[blk, blk,
                  pl.BlockSpec((H, BS, DH), lambda i: (0, i, 0)),
                  full((D, D)), full((D, D)), vec,
                  full((NN, NB)), full((NB, D)),
                  full((D, NB * R)), full((NB * R, DFF)),
                  full((DFF, D)), vec, vec, vec, vec, vec,
                  full((NB, NB * R)), full((NB * R, R)), full((R, NB * R))],
        out_specs=[pl.BlockSpec((BS, D), lambda i: (i, 0))] * 2,
        out_shape=[jax.ShapeDtypeStruct((S, D), jnp.float32)] * 2,
    )(x, h1, ctx, sW1, sW2, sb.reshape(1, D), rec, bemb, Af, Bf, WdWl,
      Wdbl.reshape(1, D), n2gl.reshape(1, D), n2bl.reshape(1, D),
      gn.reshape(1, D), bn.reshape(1, D), rep, summ, tile)


def _logits_body(h_ref, te_ref, o_ref):
    o_ref[...] = lax.dot_general(h_ref[...], te_ref[...],
                                 (((1,), (1,)), ((), ())),
                                 preferred_element_type=jnp.float32)


def _logits(hfin, token_emb):
    v = token_emb.shape[0]
    return pl.pallas_call(
        _logits_body,
        grid=(v // VB, S // BS),
        in_specs=[pl.BlockSpec((BS, D), lambda vb, s: (s, 0)),
                  pl.BlockSpec((VB, D), lambda vb, s: (vb, 0))],
        out_specs=pl.BlockSpec((BS, VB), lambda vb, s: (s, vb)),
        out_shape=jax.ShapeDtypeStruct((S, v), jnp.float32),
    )(hfin, token_emb)


def kernel(input_ids, token_emb, pos_emb, qW, qb, kW, kb, vW, vb, sW, sb,
           recipe, WdW, Wdb, n1g, n1b, n2g, n2b, ng, nb,
           basis_A, basis_B, basis_emb):
    ids = input_ids.reshape(S).astype(jnp.int32)
    g = _embed_gather(token_emb, ids)
    x, h1 = _embed_finish(g, pos_emb, n1g[0], n1b[0])

    Af = basis_A.transpose(1, 0, 2).reshape(D, NB * R)
    Bf = basis_B.reshape(NB * R, DFF)
    rep = jnp.kron(jnp.eye(NB, dtype=jnp.float32),
                   jnp.ones((1, R), jnp.float32))
    summ = jnp.tile(jnp.eye(R, dtype=jnp.float32), (NB, 1))
    tile = jnp.tile(jnp.eye(R, dtype=jnp.float32), (1, NB))

    qWh = qW.reshape(L, D, H, DH).transpose(0, 2, 1, 3)
    kWh = kW.reshape(L, D, H, DH).transpose(0, 2, 1, 3)
    vWh = vW.reshape(L, D, H, DH).transpose(0, 2, 1, 3)
    qbh = qb.reshape(L, H, 1, DH)
    kbh = kb.reshape(L, H, 1, DH)
    vbh = vb.reshape(L, H, 1, DH)

    for l in range(L):
        ctx = _attn(h1, qWh[l], kWh[l], vWh[l], qbh[l], kbh[l], vbh[l])
        gn = n1g[l + 1] if l < L - 1 else ng
        bn = n1b[l + 1] if l < L - 1 else nb
        x, h1 = _ffn(x, h1, ctx, sW[l, :D], sW[l, D:], sb[l], recipe[l],
                     basis_emb, Af, Bf, WdW[l], Wdb[l], n2g[l], n2b[l],
                     gn, bn, rep, summ, tile)

    logits = _logits(h1, token_emb)
    return logits.reshape(1, S, token_emb.shape[0])


# speedup vs baseline: 1.2873x; 1.0338x over previous
import functools

import jax
import jax.numpy as jnp
from jax import lax
from jax.experimental import pallas as pl
from jax.experimental.pallas import tpu as pltpu
from jax.experimental.pallas import tpu_sc as plsc

S, D, H, DH = 2048, 768, 12, 64
NB, R, DFF, NN, K = 32, 64, 1024, 64, 8
L = 4
BS = 256
ASB = 512
VB = 2048
EPS = 1e-5


def _lnorm(x, g, b):
    mu = jnp.mean(x, axis=-1, keepdims=True)
    var = jnp.mean((x - mu) ** 2, axis=-1, keepdims=True)
    return (x - mu) * lax.rsqrt(var + EPS) * g + b


def _embed_gather(table, ids):
    info = plsc.get_sparse_core_info()
    nw = info.num_cores * info.num_subcores
    n = ids.shape[0]
    bpw = n // nw
    d = table.shape[1]
    mesh = plsc.VectorSubcoreMesh(core_axis_name="c", subcore_axis_name="s")

    @functools.partial(
        pl.kernel, mesh=mesh,
        out_type=jax.ShapeDtypeStruct((n, d), table.dtype),
        scratch_types=[
            pltpu.VMEM((bpw,), jnp.int32),
            pltpu.VMEM((bpw, d), table.dtype),
            pltpu.SemaphoreType.DMA,
        ],
    )
    def gather_k(table_hbm, idx_hbm, out_hbm, idx_v, rows_v, sem):
        wid = lax.axis_index("s") * info.num_cores + lax.axis_index("c")
        base = wid * bpw
        pltpu.sync_copy(idx_hbm.at[pl.ds(base, bpw)], idx_v)
        pltpu.async_copy(table_hbm.at[idx_v], rows_v, sem).wait()
        pltpu.sync_copy(rows_v, out_hbm.at[pl.ds(base, bpw)])

    return gather_k(table, ids)


def _embed_finish_body(g_ref, pos_ref, g1_ref, b1_ref, x_ref, h1_ref, h1b_ref):
    x = g_ref[...] + pos_ref[...]
    x_ref[...] = x
    h1 = _lnorm(x, g1_ref[...], b1_ref[...])
    h1_ref[...] = h1
    h1b_ref[...] = h1.astype(jnp.bfloat16)


def _embed_finish(g, pos, g1, b1):
    blk = pl.BlockSpec((BS, D), lambda i: (i, 0))
    vec = pl.BlockSpec((1, D), lambda i: (0, 0))
    return pl.pallas_call(
        _embed_finish_body,
        grid=(S // BS,),
        in_specs=[blk, blk, vec, vec],
        out_specs=[pl.BlockSpec((BS, D), lambda i: (i, 0))] * 3,
        out_shape=[jax.ShapeDtypeStruct((S, D), jnp.float32),
                   jax.ShapeDtypeStruct((S, D), jnp.float32),
                   jax.ShapeDtypeStruct((S, D), jnp.bfloat16)],
    )(g, pos, g1.reshape(1, D), b1.reshape(1, D))


def _attn_body(h1_ref, qW_ref, kW_ref, vW_ref, qb_ref, kb_ref, vb_ref,
               o_ref, kk_s, vv_s):
    sblk = pl.program_id(1)

    @pl.when(sblk == 0)
    def _():
        h1 = h1_ref[...]
        kk_s[...] = (jnp.dot(h1, kW_ref[0], preferred_element_type=jnp.float32)
                     + kb_ref[0]).astype(jnp.bfloat16)
        vv_s[...] = (jnp.dot(h1, vW_ref[0], preferred_element_type=jnp.float32)
                     + vb_ref[0]).astype(jnp.bfloat16)

    q = (jnp.dot(h1_ref[pl.ds(sblk * ASB, ASB), :], qW_ref[0],
                 preferred_element_type=jnp.float32)
         + qb_ref[0]).astype(jnp.bfloat16)
    scores = lax.dot_general(q, kk_s[...], (((1,), (1,)), ((), ())),
                             preferred_element_type=jnp.float32)
    scores = scores * (1.0 / (DH ** 0.5))
    m = jnp.max(scores, axis=-1, keepdims=True)
    p = jnp.exp(scores - m)
    denom = jnp.sum(p, axis=-1, keepdims=True)
    ctx = jnp.dot(p.astype(jnp.bfloat16), vv_s[...],
                  preferred_element_type=jnp.float32) / denom
    o_ref[0] = ctx


def _attn(h1b, qWh, kWh, vWh, qbh, kbh, vbh):
    wspec = pl.BlockSpec((1, D, DH), lambda h, s: (h, 0, 0))
    bspec = pl.BlockSpec((1, 1, DH), lambda h, s: (h, 0, 0))
    return pl.pallas_call(
        _attn_body,
        grid=(H, S // ASB),
        in_specs=[pl.BlockSpec((S, D), lambda h, s: (0, 0)),
                  wspec, wspec, wspec, bspec, bspec, bspec],
        out_specs=pl.BlockSpec((1, ASB, DH), lambda h, s: (h, s, 0)),
        out_shape=jax.ShapeDtypeStruct((H, S, DH), jnp.float32),
        scratch_shapes=[pltpu.VMEM((S, DH), jnp.bfloat16),
                        pltpu.VMEM((S, DH), jnp.bfloat16)],
        compiler_params=pltpu.CompilerParams(
            dimension_semantics=("arbitrary", "arbitrary")),
    )(h1b, qWh, kWh, vWh, qbh, kbh, vbh)


def _ffn_body(x_ref, h1_ref, ctx_ref, sW1_ref, sW2_ref, sb_ref, rec_ref,
              bemb_ref, Af_ref, Bf_ref, WdW_ref, Wdb_ref, n2g_ref, n2b_ref,
              gn_ref, bn_ref, rep_ref, summ_ref, tile_ref,
              xo_ref, h1o_ref, h1o16_ref):
    h1 = h1_ref[...]
    ctx = jnp.concatenate([ctx_ref[h] for h in range(H)], axis=-1)
    query = (jnp.dot(h1, sW1_ref[...], preferred_element_type=jnp.float32)
             + jnp.dot(ctx, sW2_ref[...], preferred_element_type=jnp.float32)
             + sb_ref[...])

    rec = rec_ref[...]
    er = jnp.exp(rec - jnp.max(rec, axis=-1, keepdims=True))
    srec = er / jnp.sum(er, axis=-1, keepdims=True)
    nemb = jnp.dot(srec, bemb_ref[...], preferred_element_type=jnp.float32)
    scores = lax.dot_general(query, nemb, (((1,), (1,)), ((), ())),
                             preferred_element_type=jnp.float32)

    iota = lax.broadcasted_iota(jnp.int32, (BS, NN), 1)
    work = scores
    selmask = jnp.zeros((BS, NN), jnp.bool_)
    for _ in range(K):
        cm = jnp.max(work, axis=-1, keepdims=True)
        cand = jnp.where(work == cm, iota, NN)
        first = jnp.min(cand, axis=-1, keepdims=True)
        onehot = iota == first
        selmask = jnp.logical_or(selmask, onehot)
        work = jnp.where(onehot, -jnp.inf, work)
    gmax = jnp.max(scores, axis=-1, keepdims=True)
    wnum = jnp.where(selmask, jnp.exp(scores - gmax), 0.0)
    wd = wnum / jnp.sum(wnum, axis=-1, keepdims=True)
    tr = jnp.dot(wd, srec, preferred_element_type=jnp.float32)

    x = x_ref[...]
    h2 = _lnorm(x, n2g_ref[...], n2b_ref[...])
    u = jnp.dot(h2.astype(jnp.bfloat16), Af_ref[...],
                preferred_element_type=jnp.float32)
    t = jnp.dot(tr, rep_ref[...], preferred_element_type=jnp.float32)
    hh = jnp.dot(u * t, summ_ref[...], preferred_element_type=jnp.float32)
    hrep = jnp.dot(hh, tile_ref[...], preferred_element_type=jnp.float32)
    ff = jnp.dot((t * hrep).astype(jnp.bfloat16), Bf_ref[...],
                 preferred_element_type=jnp.float32)
    ff = ff * 0.5 * (1.0 + lax.erf(ff * (2.0 ** -0.5)))
    y = (jnp.dot(ff.astype(jnp.bfloat16), WdW_ref[...],
                 preferred_element_type=jnp.float32) + Wdb_ref[...])
    xn = x + y
    xo_ref[...] = xn
    h1o = _lnorm(xn, gn_ref[...], bn_ref[...])
    h1o_ref[...] = h1o
    h1o16_ref[...] = h1o.astype(jnp.bfloat16)


def _ffn(x, h1, ctx, sW1, sW2, sb, rec, bemb, Af, Bf, WdWl, Wdbl,
         n2gl, n2bl, gn, bn, rep, summ, tile):
    blk = pl.BlockSpec((BS, D), lambda i: (i, 0))
    vec = pl.BlockSpec((1, D), lambda i: (0, 0))
    full = lambda shape: pl.BlockSpec(shape, lambda i: (0,) * len(shape))
    return pl.pallas_call(
        _ffn_body,
        grid=(S // BS,),
        in_specs=[blk, blk,
                  pl.BlockSpec((H, BS, DH), lambda i: (0, i, 0)),
                  full((D, D)), full((D, D)), vec,
                  full((NN, NB)), full((NB, D)),
                  full((D, NB * R)), full((NB * R, DFF)),
                  full((DFF, D)), vec, vec, vec, vec, vec,
                  full((NB, NB * R)), full((NB * R, R)), full((R, NB * R))],
        out_specs=[pl.BlockSpec((BS, D), lambda i: (i, 0))] * 3,
        out_shape=[jax.ShapeDtypeStruct((S, D), jnp.float32),
                   jax.ShapeDtypeStruct((S, D), jnp.float32),
                   jax.ShapeDtypeStruct((S, D), jnp.bfloat16)],
    )(x, h1, ctx, sW1, sW2, sb.reshape(1, D), rec, bemb, Af, Bf, WdWl,
      Wdbl.reshape(1, D), n2gl.reshape(1, D), n2bl.reshape(1, D),
      gn.reshape(1, D), bn.reshape(1, D), rep, summ, tile)


def _logits_body(h_ref, te_ref, o_ref):
    o_ref[...] = lax.dot_general(h_ref[...], te_ref[...].astype(jnp.bfloat16),
                                 (((1,), (1,)), ((), ())),
                                 preferred_element_type=jnp.float32)


def _logits(hfin, token_emb):
    v = token_emb.shape[0]
    return pl.pallas_call(
        _logits_body,
        grid=(v // VB, S // BS),
        in_specs=[pl.BlockSpec((BS, D), lambda vb, s: (s, 0)),
                  pl.BlockSpec((VB, D), lambda vb, s: (vb, 0))],
        out_specs=pl.BlockSpec((BS, VB), lambda vb, s: (s, vb)),
        out_shape=jax.ShapeDtypeStruct((S, v), jnp.float32),
    )(hfin, token_emb)


def kernel(input_ids, token_emb, pos_emb, qW, qb, kW, kb, vW, vb, sW, sb,
           recipe, WdW, Wdb, n1g, n1b, n2g, n2b, ng, nb,
           basis_A, basis_B, basis_emb):
    ids = input_ids.reshape(S).astype(jnp.int32)
    g = _embed_gather(token_emb, ids)
    x, h1, h1b = _embed_finish(g, pos_emb, n1g[0], n1b[0])

    Af = basis_A.transpose(1, 0, 2).reshape(D, NB * R)
    Bf = basis_B.reshape(NB * R, DFF)
    rep = jnp.kron(jnp.eye(NB, dtype=jnp.float32),
                   jnp.ones((1, R), jnp.float32))
    summ = jnp.tile(jnp.eye(R, dtype=jnp.float32), (NB, 1))
    tile = jnp.tile(jnp.eye(R, dtype=jnp.float32), (1, NB))

    Af = Af.astype(jnp.bfloat16)
    Bf = Bf.astype(jnp.bfloat16)
    WdW16 = WdW.astype(jnp.bfloat16)
    qWh = qW.reshape(L, D, H, DH).transpose(0, 2, 1, 3).astype(jnp.bfloat16)
    kWh = kW.reshape(L, D, H, DH).transpose(0, 2, 1, 3).astype(jnp.bfloat16)
    vWh = vW.reshape(L, D, H, DH).transpose(0, 2, 1, 3).astype(jnp.bfloat16)
    qbh = qb.reshape(L, H, 1, DH)
    kbh = kb.reshape(L, H, 1, DH)
    vbh = vb.reshape(L, H, 1, DH)

    for l in range(L):
        ctx = _attn(h1b, qWh[l], kWh[l], vWh[l], qbh[l], kbh[l], vbh[l])
        gn = n1g[l + 1] if l < L - 1 else ng
        bn = n1b[l + 1] if l < L - 1 else nb
        x, h1, h1b = _ffn(x, h1, ctx, sW[l, :D], sW[l, D:], sb[l], recipe[l],
                          basis_emb, Af, Bf, WdW16[l], Wdb[l], n2g[l], n2b[l],
                          gn, bn, rep, summ, tile)

    logits = _logits(h1b, token_emb)
    return logits.reshape(1, S, token_emb.shape[0])


# speedup vs baseline: 1.3854x; 1.0761x over previous
import functools

import jax
import jax.numpy as jnp
from jax import lax
from jax.experimental import pallas as pl
from jax.experimental.pallas import tpu as pltpu
from jax.experimental.pallas import tpu_sc as plsc

S, D, H, DH = 2048, 768, 12, 64
NB, R, DFF, NN, K = 32, 64, 1024, 64, 8
L = 4
BS = 256
ASB = 512
VB = 2048
EPS = 1e-5


def _lnorm(x, g, b):
    mu = jnp.mean(x, axis=-1, keepdims=True)
    var = jnp.mean((x - mu) ** 2, axis=-1, keepdims=True)
    return (x - mu) * lax.rsqrt(var + EPS) * g + b


def _embed_gather(table, ids):
    info = plsc.get_sparse_core_info()
    nw = info.num_cores * info.num_subcores
    n = ids.shape[0]
    bpw = n // nw
    d = table.shape[1]
    mesh = plsc.VectorSubcoreMesh(core_axis_name="c", subcore_axis_name="s")

    @functools.partial(
        pl.kernel, mesh=mesh,
        out_type=jax.ShapeDtypeStruct((n, d), table.dtype),
        scratch_types=[
            pltpu.VMEM((bpw,), jnp.int32),
            pltpu.VMEM((bpw, d), table.dtype),
            pltpu.SemaphoreType.DMA,
        ],
    )
    def gather_k(table_hbm, idx_hbm, out_hbm, idx_v, rows_v, sem):
        wid = lax.axis_index("s") * info.num_cores + lax.axis_index("c")
        base = wid * bpw
        pltpu.sync_copy(idx_hbm.at[pl.ds(base, bpw)], idx_v)
        pltpu.async_copy(table_hbm.at[idx_v], rows_v, sem).wait()
        pltpu.sync_copy(rows_v, out_hbm.at[pl.ds(base, bpw)])

    return gather_k(table, ids)


def _embed_finish_body(g_ref, pos_ref, g1_ref, b1_ref, x_ref, h1_ref, h1b_ref):
    x = g_ref[...] + pos_ref[...]
    x_ref[...] = x
    h1 = _lnorm(x, g1_ref[...], b1_ref[...])
    h1_ref[...] = h1
    h1b_ref[...] = h1.astype(jnp.bfloat16)


def _embed_finish(g, pos, g1, b1):
    blk = pl.BlockSpec((BS, D), lambda i: (i, 0))
    vec = pl.BlockSpec((1, D), lambda i: (0, 0))
    return pl.pallas_call(
        _embed_finish_body,
        grid=(S // BS,),
        in_specs=[blk, blk, vec, vec],
        out_specs=[pl.BlockSpec((BS, D), lambda i: (i, 0))] * 3,
        out_shape=[jax.ShapeDtypeStruct((S, D), jnp.float32),
                   jax.ShapeDtypeStruct((S, D), jnp.float32),
                   jax.ShapeDtypeStruct((S, D), jnp.bfloat16)],
    )(g, pos, g1.reshape(1, D), b1.reshape(1, D))


def _qkv_body(h1_ref, qW_ref, kW_ref, vW_ref, qb_ref, kb_ref, vb_ref,
              q_ref, k_ref, v_ref):
    h1 = h1_ref[...]
    q = (jnp.dot(h1, qW_ref[...], preferred_element_type=jnp.float32)
         + qb_ref[...]).astype(jnp.bfloat16)
    k = (jnp.dot(h1, kW_ref[...], preferred_element_type=jnp.float32)
         + kb_ref[...]).astype(jnp.bfloat16)
    v = (jnp.dot(h1, vW_ref[...], preferred_element_type=jnp.float32)
         + vb_ref[...]).astype(jnp.bfloat16)
    for h in range(H):
        q_ref[h] = q[:, h * DH:(h + 1) * DH]
        k_ref[h] = k[:, h * DH:(h + 1) * DH]
        v_ref[h] = v[:, h * DH:(h + 1) * DH]


def _qkv(h1b, qWl, kWl, vWl, qbl, kbl, vbl):
    blk = pl.BlockSpec((BS, D), lambda i: (i, 0))
    wfull = pl.BlockSpec((D, D), lambda i: (0, 0))
    vec = pl.BlockSpec((1, D), lambda i: (0, 0))
    return pl.pallas_call(
        _qkv_body,
        grid=(S // BS,),
        in_specs=[blk, wfull, wfull, wfull, vec, vec, vec],
        out_specs=[pl.BlockSpec((H, BS, DH), lambda i: (0, i, 0))] * 3,
        out_shape=[jax.ShapeDtypeStruct((H, S, DH), jnp.bfloat16)] * 3,
    )(h1b, qWl, kWl, vWl, qbl.reshape(1, D), kbl.reshape(1, D),
      vbl.reshape(1, D))


def _attn_body(q_ref, k_ref, v_ref, o_ref):
    scores = lax.dot_general(q_ref[0], k_ref[0], (((1,), (1,)), ((), ())),
                             preferred_element_type=jnp.float32)
    scores = scores * (1.0 / (DH ** 0.5))
    m = jnp.max(scores, axis=-1, keepdims=True)
    p = jnp.exp(scores - m)
    denom = jnp.sum(p, axis=-1, keepdims=True)
    ctx = jnp.dot(p.astype(jnp.bfloat16), v_ref[0],
                  preferred_element_type=jnp.float32) / denom
    o_ref[0] = ctx


def _attn(q, k, v):
    hspec = pl.BlockSpec((1, S, DH), lambda h: (h, 0, 0))
    return pl.pallas_call(
        _attn_body,
        grid=(H,),
        in_specs=[hspec, hspec, hspec],
        out_specs=hspec,
        out_shape=jax.ShapeDtypeStruct((H, S, DH), jnp.float32),
    )(q, k, v)


def _ffn_body(x_ref, h1_ref, ctx_ref, sW1_ref, sW2_ref, sb_ref, rec_ref,
              bemb_ref, Af_ref, Bf_ref, WdW_ref, Wdb_ref, n2g_ref, n2b_ref,
              gn_ref, bn_ref, rep_ref, summ_ref, tile_ref,
              xo_ref, h1o_ref, h1o16_ref):
    h1 = h1_ref[...]
    query = (jnp.dot(h1, sW1_ref[...], preferred_element_type=jnp.float32)
             + sb_ref[...])
    for h in range(H):
        query = query + jnp.dot(ctx_ref[h], sW2_ref[h],
                                preferred_element_type=jnp.float32)

    rec = rec_ref[...]
    er = jnp.exp(rec - jnp.max(rec, axis=-1, keepdims=True))
    srec = er / jnp.sum(er, axis=-1, keepdims=True)
    nemb = jnp.dot(srec, bemb_ref[...], preferred_element_type=jnp.float32)
    scores = lax.dot_general(query, nemb, (((1,), (1,)), ((), ())),
                             preferred_element_type=jnp.float32)

    iota = lax.broadcasted_iota(jnp.int32, (BS, NN), 1)
    work = scores
    selmask = jnp.zeros((BS, NN), jnp.bool_)
    for _ in range(K):
        cm = jnp.max(work, axis=-1, keepdims=True)
        cand = jnp.where(work == cm, iota, NN)
        first = jnp.min(cand, axis=-1, keepdims=True)
        onehot = iota == first
        selmask = jnp.logical_or(selmask, onehot)
        work = jnp.where(onehot, -jnp.inf, work)
    gmax = jnp.max(scores, axis=-1, keepdims=True)
    wnum = jnp.where(selmask, jnp.exp(scores - gmax), 0.0)
    wd = wnum / jnp.sum(wnum, axis=-1, keepdims=True)
    tr = jnp.dot(wd, srec, preferred_element_type=jnp.float32)

    x = x_ref[...]
    h2 = _lnorm(x, n2g_ref[...], n2b_ref[...])
    u = jnp.dot(h2.astype(jnp.bfloat16), Af_ref[...],
                preferred_element_type=jnp.float32)
    t = jnp.dot(tr, rep_ref[...], preferred_element_type=jnp.float32)
    hh = jnp.dot(u * t, summ_ref[...], preferred_element_type=jnp.float32)
    hrep = jnp.dot(hh, tile_ref[...], preferred_element_type=jnp.float32)
    ff = jnp.dot((t * hrep).astype(jnp.bfloat16), Bf_ref[...],
                 preferred_element_type=jnp.float32)
    ff = ff * 0.5 * (1.0 + lax.erf(ff * (2.0 ** -0.5)))
    y = (jnp.dot(ff.astype(jnp.bfloat16), WdW_ref[...],
                 preferred_element_type=jnp.float32) + Wdb_ref[...])
    xn = x + y
    xo_ref[...] = xn
    h1o = _lnorm(xn, gn_ref[...], bn_ref[...])
    h1o_ref[...] = h1o
    h1o16_ref[...] = h1o.astype(jnp.bfloat16)


def _ffn(x, h1, ctx, sW1, sW2, sb, rec, bemb, Af, Bf, WdWl, Wdbl,
         n2gl, n2bl, gn, bn, rep, summ, tile):
    blk = pl.BlockSpec((BS, D), lambda i: (i, 0))
    vec = pl.BlockSpec((1, D), lambda i: (0, 0))
    full = lambda shape: pl.BlockSpec(shape, lambda i: (0,) * len(shape))
    return pl.pallas_call(
        _ffn_body,
        grid=(S // BS,),
        in_specs=[blk, blk,
                  pl.BlockSpec((H, BS, DH), lambda i: (0, i, 0)),
                  full((D, D)), full((H, DH, D)), vec,
                  full((NN, NB)), full((NB, D)),
                  full((D, NB * R)), full((NB * R, DFF)),
                  full((DFF, D)), vec, vec, vec, vec, vec,
                  full((NB, NB * R)), full((NB * R, R)), full((R, NB * R))],
        out_specs=[pl.BlockSpec((BS, D), lambda i: (i, 0))] * 3,
        out_shape=[jax.ShapeDtypeStruct((S, D), jnp.float32),
                   jax.ShapeDtypeStruct((S, D), jnp.float32),
                   jax.ShapeDtypeStruct((S, D), jnp.bfloat16)],
    )(x, h1, ctx, sW1, sW2, sb.reshape(1, D), rec, bemb, Af, Bf, WdWl,
      Wdbl.reshape(1, D), n2gl.reshape(1, D), n2bl.reshape(1, D),
      gn.reshape(1, D), bn.reshape(1, D), rep, summ, tile)


def _logits_body(h_ref, te_ref, o_ref):
    o_ref[...] = lax.dot_general(h_ref[...], te_ref[...].astype(jnp.bfloat16),
                                 (((1,), (1,)), ((), ())),
                                 preferred_element_type=jnp.float32)


def _logits(hfin, token_emb):
    v = token_emb.shape[0]
    return pl.pallas_call(
        _logits_body,
        grid=(v // VB, S // BS),
        in_specs=[pl.BlockSpec((BS, D), lambda vb, s: (s, 0)),
                  pl.BlockSpec((VB, D), lambda vb, s: (vb, 0))],
        out_specs=pl.BlockSpec((BS, VB), lambda vb, s: (s, vb)),
        out_shape=jax.ShapeDtypeStruct((S, v), jnp.float32),
    )(hfin, token_emb)


def kernel(input_ids, token_emb, pos_emb, qW, qb, kW, kb, vW, vb, sW, sb,
           recipe, WdW, Wdb, n1g, n1b, n2g, n2b, ng, nb,
           basis_A, basis_B, basis_emb):
    ids = input_ids.reshape(S).astype(jnp.int32)
    g = _embed_gather(token_emb, ids)
    x, h1, h1b = _embed_finish(g, pos_emb, n1g[0], n1b[0])

    Af = basis_A.transpose(1, 0, 2).reshape(D, NB * R)
    Bf = basis_B.reshape(NB * R, DFF)
    rep = jnp.kron(jnp.eye(NB, dtype=jnp.float32),
                   jnp.ones((1, R), jnp.float32))
    summ = jnp.tile(jnp.eye(R, dtype=jnp.float32), (NB, 1))
    tile = jnp.tile(jnp.eye(R, dtype=jnp.float32), (1, NB))

    Af = Af.astype(jnp.bfloat16)
    Bf = Bf.astype(jnp.bfloat16)
    WdW16 = WdW.astype(jnp.bfloat16)
    qW16 = qW.astype(jnp.bfloat16)
    kW16 = kW.astype(jnp.bfloat16)
    vW16 = vW.astype(jnp.bfloat16)
    sW2h = sW[:, D:].reshape(L, H, DH, D)

    for l in range(L):
        q, k, v = _qkv(h1b, qW16[l], kW16[l], vW16[l], qb[l], kb[l], vb[l])
        ctx = _attn(q, k, v)
        gn = n1g[l + 1] if l < L - 1 else ng
        bn = n1b[l + 1] if l < L - 1 else nb
        x, h1, h1b = _ffn(x, h1, ctx, sW[l, :D], sW2h[l], sb[l], recipe[l],
                          basis_emb, Af, Bf, WdW16[l], Wdb[l], n2g[l], n2b[l],
                          gn, bn, rep, summ, tile)

    logits = _logits(h1b, token_emb)
    return logits.reshape(1, S, token_emb.shape[0])


# speedup vs baseline: 1.7127x; 1.2363x over previous
import functools

import jax
import jax.numpy as jnp
from jax import lax
from jax.experimental import pallas as pl
from jax.experimental.pallas import tpu as pltpu
from jax.experimental.pallas import tpu_sc as plsc

S, D, H, DH = 2048, 768, 12, 64
NB, R, DFF, NN, K = 32, 64, 1024, 64, 8
L = 4
BS = 256
ASB = 512
VB = 2048
EPS = 1e-5


def _lnorm(x, g, b):
    mu = jnp.mean(x, axis=-1, keepdims=True)
    var = jnp.mean((x - mu) ** 2, axis=-1, keepdims=True)
    return (x - mu) * lax.rsqrt(var + EPS) * g + b


def _embed_gather(table, ids):
    info = plsc.get_sparse_core_info()
    nw = info.num_cores * info.num_subcores
    n = ids.shape[0]
    bpw = n // nw
    d = table.shape[1]
    mesh = plsc.VectorSubcoreMesh(core_axis_name="c", subcore_axis_name="s")

    @functools.partial(
        pl.kernel, mesh=mesh,
        out_type=jax.ShapeDtypeStruct((n, d), table.dtype),
        scratch_types=[
            pltpu.VMEM((bpw,), jnp.int32),
            pltpu.VMEM((bpw, d), table.dtype),
            pltpu.SemaphoreType.DMA,
        ],
    )
    def gather_k(table_hbm, idx_hbm, out_hbm, idx_v, rows_v, sem):
        wid = lax.axis_index("s") * info.num_cores + lax.axis_index("c")
        base = wid * bpw
        pltpu.sync_copy(idx_hbm.at[pl.ds(base, bpw)], idx_v)
        pltpu.async_copy(table_hbm.at[idx_v], rows_v, sem).wait()
        pltpu.sync_copy(rows_v, out_hbm.at[pl.ds(base, bpw)])

    return gather_k(table, ids)


def _embed_finish_body(g_ref, pos_ref, g1_ref, b1_ref, x_ref, h1_ref, h1b_ref):
    x = g_ref[...] + pos_ref[...]
    x_ref[...] = x
    h1 = _lnorm(x, g1_ref[...], b1_ref[...])
    h1_ref[...] = h1
    h1b_ref[...] = h1.astype(jnp.bfloat16)


def _embed_finish(g, pos, g1, b1):
    blk = pl.BlockSpec((BS, D), lambda i: (i, 0))
    vec = pl.BlockSpec((1, D), lambda i: (0, 0))
    return pl.pallas_call(
        _embed_finish_body,
        grid=(S // BS,),
        in_specs=[blk, blk, vec, vec],
        out_specs=[pl.BlockSpec((BS, D), lambda i: (i, 0))] * 3,
        out_shape=[jax.ShapeDtypeStruct((S, D), jnp.float32),
                   jax.ShapeDtypeStruct((S, D), jnp.float32),
                   jax.ShapeDtypeStruct((S, D), jnp.bfloat16)],
    )(g, pos, g1.reshape(1, D), b1.reshape(1, D))


def _qkv_body(h1_ref, qW_ref, kW_ref, vW_ref, qb_ref, kb_ref, vb_ref,
              q_ref, k_ref, v_ref):
    h1 = h1_ref[...]
    q = (jnp.dot(h1, qW_ref[...], preferred_element_type=jnp.float32)
         + qb_ref[...]).astype(jnp.bfloat16)
    k = (jnp.dot(h1, kW_ref[...], preferred_element_type=jnp.float32)
         + kb_ref[...]).astype(jnp.bfloat16)
    v = (jnp.dot(h1, vW_ref[...], preferred_element_type=jnp.float32)
         + vb_ref[...]).astype(jnp.bfloat16)
    for h in range(H):
        q_ref[h] = q[:, h * DH:(h + 1) * DH]
        k_ref[h] = k[:, h * DH:(h + 1) * DH]
        v_ref[h] = v[:, h * DH:(h + 1) * DH]


def _qkv(h1b, qWl, kWl, vWl, qbl, kbl, vbl):
    blk = pl.BlockSpec((BS, D), lambda i: (i, 0))
    wfull = pl.BlockSpec((D, D), lambda i: (0, 0))
    vec = pl.BlockSpec((1, D), lambda i: (0, 0))
    return pl.pallas_call(
        _qkv_body,
        grid=(S // BS,),
        in_specs=[blk, wfull, wfull, wfull, vec, vec, vec],
        out_specs=[pl.BlockSpec((H, BS, DH), lambda i: (0, i, 0))] * 3,
        out_shape=[jax.ShapeDtypeStruct((H, S, DH), jnp.bfloat16)] * 3,
    )(h1b, qWl, kWl, vWl, qbl.reshape(1, D), kbl.reshape(1, D),
      vbl.reshape(1, D))


def _attn_body(q_ref, k_ref, v_ref, o_ref):
    k = k_ref[0]
    v = v_ref[0]
    ones = jnp.ones((S, DH), jnp.bfloat16)
    for c in range(S // ASB):
        rows = pl.ds(c * ASB, ASB)
        scores = lax.dot_general(q_ref[0, rows, :], k,
                                 (((1,), (1,)), ((), ())),
                                 preferred_element_type=jnp.float32)
        scores = scores * (1.0 / (DH ** 0.5))
        m = jnp.max(scores, axis=-1, keepdims=True)
        p16 = jnp.exp(scores - m).astype(jnp.bfloat16)
        denom = jnp.dot(p16, ones, preferred_element_type=jnp.float32)[:, :1]
        ctx = jnp.dot(p16, v, preferred_element_type=jnp.float32) / denom
        o_ref[0, rows, :] = ctx


def _attn(q, k, v):
    hspec = pl.BlockSpec((1, S, DH), lambda h: (h, 0, 0))
    return pl.pallas_call(
        _attn_body,
        grid=(H,),
        in_specs=[hspec, hspec, hspec],
        out_specs=hspec,
        out_shape=jax.ShapeDtypeStruct((H, S, DH), jnp.float32),
    )(q, k, v)


def _ffn_body(x_ref, h1_ref, ctx_ref, sW1_ref, sW2_ref, sb_ref, rec_ref,
              bemb_ref, Af_ref, Bf_ref, WdW_ref, Wdb_ref, n2g_ref, n2b_ref,
              gn_ref, bn_ref, rep_ref, summ_ref, tile_ref,
              xo_ref, h1o_ref, h1o16_ref):
    h1 = h1_ref[...]
    ctx = jnp.concatenate([ctx_ref[h] for h in range(H)], axis=-1)
    query = (jnp.dot(h1, sW1_ref[...], preferred_element_type=jnp.float32)
             + jnp.dot(ctx, sW2_ref[...], preferred_element_type=jnp.float32)
             + sb_ref[...])

    rec = rec_ref[...]
    er = jnp.exp(rec - jnp.max(rec, axis=-1, keepdims=True))
    srec = er / jnp.sum(er, axis=-1, keepdims=True)
    nemb = jnp.dot(srec, bemb_ref[...], preferred_element_type=jnp.float32)
    scores = lax.dot_general(query, nemb, (((1,), (1,)), ((), ())),
                             preferred_element_type=jnp.float32)

    iota = lax.broadcasted_iota(jnp.int32, (BS, NN), 1)
    work = scores
    selmask = jnp.zeros((BS, NN), jnp.bool_)
    for _ in range(K):
        cm = jnp.max(work, axis=-1, keepdims=True)
        cand = jnp.where(work == cm, iota, NN)
        first = jnp.min(cand, axis=-1, keepdims=True)
        onehot = iota == first
        selmask = jnp.logical_or(selmask, onehot)
        work = jnp.where(onehot, -jnp.inf, work)
    gmax = jnp.max(scores, axis=-1, keepdims=True)
    wnum = jnp.where(selmask, jnp.exp(scores - gmax), 0.0)
    wd = wnum / jnp.sum(wnum, axis=-1, keepdims=True)
    tr = jnp.dot(wd, srec, preferred_element_type=jnp.float32)

    x = x_ref[...]
    h2 = _lnorm(x, n2g_ref[...], n2b_ref[...])
    u = jnp.dot(h2.astype(jnp.bfloat16), Af_ref[...],
                preferred_element_type=jnp.float32)
    t = jnp.dot(tr, rep_ref[...], preferred_element_type=jnp.float32)
    hh = jnp.dot(u * t, summ_ref[...], preferred_element_type=jnp.float32)
    hrep = jnp.dot(hh, tile_ref[...], preferred_element_type=jnp.float32)
    ff = jnp.dot((t * hrep).astype(jnp.bfloat16), Bf_ref[...],
                 preferred_element_type=jnp.float32)
    ff = ff * 0.5 * (1.0 + lax.erf(ff * (2.0 ** -0.5)))
    y = (jnp.dot(ff.astype(jnp.bfloat16), WdW_ref[...],
                 preferred_element_type=jnp.float32) + Wdb_ref[...])
    xn = x + y
    xo_ref[...] = xn
    h1o = _lnorm(xn, gn_ref[...], bn_ref[...])
    h1o_ref[...] = h1o
    h1o16_ref[...] = h1o.astype(jnp.bfloat16)


def _ffn(x, h1, ctx, sW1, sW2, sb, rec, bemb, Af, Bf, WdWl, Wdbl,
         n2gl, n2bl, gn, bn, rep, summ, tile):
    blk = pl.BlockSpec((BS, D), lambda i: (i, 0))
    vec = pl.BlockSpec((1, D), lambda i: (0, 0))
    full = lambda shape: pl.BlockSpec(shape, lambda i: (0,) * len(shape))
    return pl.pallas_call(
        _ffn_body,
        grid=(S // BS,),
        in_specs=[blk, blk,
                  pl.BlockSpec((H, BS, DH), lambda i: (0, i, 0)),
                  full((D, D)), full((D, D)), vec,
                  full((NN, NB)), full((NB, D)),
                  full((D, NB * R)), full((NB * R, DFF)),
                  full((DFF, D)), vec, vec, vec, vec, vec,
                  full((NB, NB * R)), full((NB * R, R)), full((R, NB * R))],
        out_specs=[pl.BlockSpec((BS, D), lambda i: (i, 0))] * 3,
        out_shape=[jax.ShapeDtypeStruct((S, D), jnp.float32),
                   jax.ShapeDtypeStruct((S, D), jnp.float32),
                   jax.ShapeDtypeStruct((S, D), jnp.bfloat16)],
    )(x, h1, ctx, sW1, sW2, sb.reshape(1, D), rec, bemb, Af, Bf, WdWl,
      Wdbl.reshape(1, D), n2gl.reshape(1, D), n2bl.reshape(1, D),
      gn.reshape(1, D), bn.reshape(1, D), rep, summ, tile)


def _logits_body(h_ref, te_ref, o_ref):
    o_ref[...] = lax.dot_general(h_ref[...], te_ref[...].astype(jnp.bfloat16),
                                 (((1,), (1,)), ((), ())),
                                 preferred_element_type=jnp.float32)


def _logits(hfin, token_emb):
    v = token_emb.shape[0]
    return pl.pallas_call(
        _logits_body,
        grid=(v // VB, S // BS),
        in_specs=[pl.BlockSpec((BS, D), lambda vb, s: (s, 0)),
                  pl.BlockSpec((VB, D), lambda vb, s: (vb, 0))],
        out_specs=pl.BlockSpec((BS, VB), lambda vb, s: (s, vb)),
        out_shape=jax.ShapeDtypeStruct((S, v), jnp.float32),
    )(hfin, token_emb)


def kernel(input_ids, token_emb, pos_emb, qW, qb, kW, kb, vW, vb, sW, sb,
           recipe, WdW, Wdb, n1g, n1b, n2g, n2b, ng, nb,
           basis_A, basis_B, basis_emb):
    ids = input_ids.reshape(S).astype(jnp.int32)
    g = _embed_gather(token_emb, ids)
    x, h1, h1b = _embed_finish(g, pos_emb, n1g[0], n1b[0])

    Af = basis_A.transpose(1, 0, 2).reshape(D, NB * R)
    Bf = basis_B.reshape(NB * R, DFF)
    rep = jnp.kron(jnp.eye(NB, dtype=jnp.float32),
                   jnp.ones((1, R), jnp.float32))
    summ = jnp.tile(jnp.eye(R, dtype=jnp.float32), (NB, 1))
    tile = jnp.tile(jnp.eye(R, dtype=jnp.float32), (1, NB))

    Af = Af.astype(jnp.bfloat16)
    Bf = Bf.astype(jnp.bfloat16)
    WdW16 = WdW.astype(jnp.bfloat16)
    qW16 = qW.astype(jnp.bfloat16)
    kW16 = kW.astype(jnp.bfloat16)
    vW16 = vW.astype(jnp.bfloat16)
    for l in range(L):
        q, k, v = _qkv(h1b, qW16[l], kW16[l], vW16[l], qb[l], kb[l], vb[l])
        ctx = _attn(q, k, v)
        gn = n1g[l + 1] if l < L - 1 else ng
        bn = n1b[l + 1] if l < L - 1 else nb
        x, h1, h1b = _ffn(x, h1, ctx, sW[l, :D], sW[l, D:], sb[l], recipe[l],
                          basis_emb, Af, Bf, WdW16[l], Wdb[l], n2g[l], n2b[l],
                          gn, bn, rep, summ, tile)

    logits = _logits(h1b, token_emb)
    return logits.reshape(1, S, token_emb.shape[0])


# speedup vs baseline: 1.8924x; 1.1049x over previous
import functools

import jax
import jax.numpy as jnp
from jax import lax
from jax.experimental import pallas as pl
from jax.experimental.pallas import tpu as pltpu
from jax.experimental.pallas import tpu_sc as plsc

S, D, H, DH = 2048, 768, 12, 64
NB, R, DFF, NN, K = 32, 64, 1024, 64, 8
L = 4
BS = 256
ASB = 512
VB = 2048
EPS = 1e-5


def _lnorm(x, g, b):
    mu = jnp.mean(x, axis=-1, keepdims=True)
    var = jnp.mean((x - mu) ** 2, axis=-1, keepdims=True)
    return (x - mu) * lax.rsqrt(var + EPS) * g + b


def _embed_gather(table, ids):
    info = plsc.get_sparse_core_info()
    nw = info.num_cores * info.num_subcores
    n = ids.shape[0]
    bpw = n // nw
    d = table.shape[1]
    mesh = plsc.VectorSubcoreMesh(core_axis_name="c", subcore_axis_name="s")

    @functools.partial(
        pl.kernel, mesh=mesh,
        out_type=jax.ShapeDtypeStruct((n, d), table.dtype),
        scratch_types=[
            pltpu.VMEM((bpw,), jnp.int32),
            pltpu.VMEM((bpw, d), table.dtype),
            pltpu.SemaphoreType.DMA,
        ],
    )
    def gather_k(table_hbm, idx_hbm, out_hbm, idx_v, rows_v, sem):
        wid = lax.axis_index("s") * info.num_cores + lax.axis_index("c")
        base = wid * bpw
        pltpu.sync_copy(idx_hbm.at[pl.ds(base, bpw)], idx_v)
        pltpu.async_copy(table_hbm.at[idx_v], rows_v, sem).wait()
        pltpu.sync_copy(rows_v, out_hbm.at[pl.ds(base, bpw)])

    return gather_k(table, ids)


def _qkv_emit(h1b, qW_ref, kW_ref, vW_ref, qb_ref, kb_ref, vb_ref,
              q_ref, k_ref, vaug_ref):
    q = (jnp.dot(h1b, qW_ref[...], preferred_element_type=jnp.float32)
         + qb_ref[...]).astype(jnp.bfloat16)
    k = (jnp.dot(h1b, kW_ref[...], preferred_element_type=jnp.float32)
         + kb_ref[...]).astype(jnp.bfloat16)
    v = (jnp.dot(h1b, vW_ref[...], preferred_element_type=jnp.float32)
         + vb_ref[...]).astype(jnp.bfloat16)
    ones = jnp.ones((h1b.shape[0], DH), jnp.bfloat16)
    for h in range(H):
        q_ref[h] = q[:, h * DH:(h + 1) * DH]
        k_ref[h] = k[:, h * DH:(h + 1) * DH]
        vaug_ref[h] = jnp.concatenate([v[:, h * DH:(h + 1) * DH], ones],
                                      axis=-1)


def _embed_finish_body(g_ref, pos_ref, g1_ref, b1_ref,
                       qW_ref, kW_ref, vW_ref, qb_ref, kb_ref, vb_ref,
                       x_ref, h1_ref, q_ref, k_ref, vaug_ref):
    x = g_ref[...] + pos_ref[...]
    x_ref[...] = x
    h1 = _lnorm(x, g1_ref[...], b1_ref[...])
    h1_ref[...] = h1
    _qkv_emit(h1.astype(jnp.bfloat16), qW_ref, kW_ref, vW_ref,
              qb_ref, kb_ref, vb_ref, q_ref, k_ref, vaug_ref)


def _embed_finish(g, pos, g1, b1, qWl, kWl, vWl, qbl, kbl, vbl):
    blk = pl.BlockSpec((BS, D), lambda i: (i, 0))
    vec = pl.BlockSpec((1, D), lambda i: (0, 0))
    wfull = pl.BlockSpec((D, D), lambda i: (0, 0))
    hspec = pl.BlockSpec((H, BS, DH), lambda i: (0, i, 0))
    haspec = pl.BlockSpec((H, BS, 2 * DH), lambda i: (0, i, 0))
    return pl.pallas_call(
        _embed_finish_body,
        grid=(S // BS,),
        in_specs=[blk, blk, vec, vec, wfull, wfull, wfull, vec, vec, vec],
        out_specs=[pl.BlockSpec((BS, D), lambda i: (i, 0))] * 2
                  + [hspec, hspec, haspec],
        out_shape=[jax.ShapeDtypeStruct((S, D), jnp.float32),
                   jax.ShapeDtypeStruct((S, D), jnp.float32),
                   jax.ShapeDtypeStruct((H, S, DH), jnp.bfloat16),
                   jax.ShapeDtypeStruct((H, S, DH), jnp.bfloat16),
                   jax.ShapeDtypeStruct((H, S, 2 * DH), jnp.bfloat16)],
    )(g, pos, g1.reshape(1, D), b1.reshape(1, D), qWl, kWl, vWl,
      qbl.reshape(1, D), kbl.reshape(1, D), vbl.reshape(1, D))


def _attn_body(q_ref, k_ref, vaug_ref, o_ref):
    k = k_ref[0]
    vaug = vaug_ref[0]
    for c in range(S // ASB):
        rows = pl.ds(c * ASB, ASB)
        scores = lax.dot_general(q_ref[0, rows, :], k,
                                 (((1,), (1,)), ((), ())),
                                 preferred_element_type=jnp.float32)
        scores = scores * (1.0 / (DH ** 0.5))
        m = jnp.max(scores, axis=-1, keepdims=True)
        p16 = jnp.exp(scores - m).astype(jnp.bfloat16)
        ctxd = jnp.dot(p16, vaug, preferred_element_type=jnp.float32)
        ctx = ctxd[:, :DH] / ctxd[:, DH:DH + 1]
        o_ref[0, rows, :] = ctx


def _attn(q, k, vaug):
    hspec = pl.BlockSpec((1, S, DH), lambda h: (h, 0, 0))
    haspec = pl.BlockSpec((1, S, 2 * DH), lambda h: (h, 0, 0))
    return pl.pallas_call(
        _attn_body,
        grid=(H,),
        in_specs=[hspec, hspec, haspec],
        out_specs=hspec,
        out_shape=jax.ShapeDtypeStruct((H, S, DH), jnp.float32),
    )(q, k, vaug)


def _ffn_body(with_qkv, x_ref, h1_ref, ctx_ref, sW1_ref, sW2_ref, sb_ref,
              rec_ref, bemb_ref, Af_ref, Bf_ref, WdW_ref, Wdb_ref,
              n2g_ref, n2b_ref, gn_ref, bn_ref, rep_ref, summ_ref, tile_ref,
              *rest):
    if with_qkv:
        (qW_ref, kW_ref, vW_ref, qb_ref, kb_ref, vb_ref,
         xo_ref, h1o_ref, q_ref, k_ref, vaug_ref) = rest
    else:
        (h1o16_ref,) = rest
    h1 = h1_ref[...]
    ctx = jnp.concatenate([ctx_ref[h] for h in range(H)], axis=-1)
    query = (jnp.dot(h1, sW1_ref[...], preferred_element_type=jnp.float32)
             + jnp.dot(ctx, sW2_ref[...], preferred_element_type=jnp.float32)
             + sb_ref[...])

    rec = rec_ref[...]
    er = jnp.exp(rec - jnp.max(rec, axis=-1, keepdims=True))
    srec = er / jnp.sum(er, axis=-1, keepdims=True)
    nemb = jnp.dot(srec, bemb_ref[...], preferred_element_type=jnp.float32)
    scores = lax.dot_general(query, nemb, (((1,), (1,)), ((), ())),
                             preferred_element_type=jnp.float32)

    iota = lax.broadcasted_iota(jnp.int32, (BS, NN), 1)
    work = scores
    selmask = jnp.zeros((BS, NN), jnp.bool_)
    for _ in range(K):
        cm = jnp.max(work, axis=-1, keepdims=True)
        cand = jnp.where(work == cm, iota, NN)
        first = jnp.min(cand, axis=-1, keepdims=True)
        onehot = iota == first
        selmask = jnp.logical_or(selmask, onehot)
        work = jnp.where(onehot, -jnp.inf, work)
    gmax = jnp.max(scores, axis=-1, keepdims=True)
    wnum = jnp.where(selmask, jnp.exp(scores - gmax), 0.0)
    wd = wnum / jnp.sum(wnum, axis=-1, keepdims=True)
    tr = jnp.dot(wd, srec, preferred_element_type=jnp.float32)

    x = x_ref[...]
    h2 = _lnorm(x, n2g_ref[...], n2b_ref[...])
    u = jnp.dot(h2.astype(jnp.bfloat16), Af_ref[...],
                preferred_element_type=jnp.float32)
    t = jnp.dot(tr, rep_ref[...], preferred_element_type=jnp.float32)
    hh = jnp.dot(u * t, summ_ref[...], preferred_element_type=jnp.float32)
    hrep = jnp.dot(hh, tile_ref[...], preferred_element_type=jnp.float32)
    ff = jnp.dot((t * hrep).astype(jnp.bfloat16), Bf_ref[...],
                 preferred_element_type=jnp.float32)
    ff = ff * 0.5 * (1.0 + lax.erf(ff * (2.0 ** -0.5)))
    y = (jnp.dot(ff.astype(jnp.bfloat16), WdW_ref[...],
                 preferred_element_type=jnp.float32) + Wdb_ref[...])
    xn = x + y
    h1o = _lnorm(xn, gn_ref[...], bn_ref[...])
    if with_qkv:
        xo_ref[...] = xn
        h1o_ref[...] = h1o
        _qkv_emit(h1o.astype(jnp.bfloat16), qW_ref, kW_ref, vW_ref,
                  qb_ref, kb_ref, vb_ref, q_ref, k_ref, vaug_ref)
    else:
        h1o16_ref[...] = h1o.astype(jnp.bfloat16)


def _ffn(x, h1, ctx, sW1, sW2, sb, rec, bemb, Af, Bf, WdWl, Wdbl,
         n2gl, n2bl, gn, bn, rep, summ, tile, qkvw=None):
    blk = pl.BlockSpec((BS, D), lambda i: (i, 0))
    vec = pl.BlockSpec((1, D), lambda i: (0, 0))
    full = lambda shape: pl.BlockSpec(shape, lambda i: (0,) * len(shape))
    in_specs = [blk, blk,
                pl.BlockSpec((H, BS, DH), lambda i: (0, i, 0)),
                full((D, D)), full((D, D)), vec,
                full((NN, NB)), full((NB, D)),
                full((D, NB * R)), full((NB * R, DFF)),
                full((DFF, D)), vec, vec, vec, vec, vec,
                full((NB, NB * R)), full((NB * R, R)), full((R, NB * R))]
    args = [x, h1, ctx, sW1, sW2, sb.reshape(1, D), rec, bemb, Af, Bf, WdWl,
            Wdbl.reshape(1, D), n2gl.reshape(1, D), n2bl.reshape(1, D),
            gn.reshape(1, D), bn.reshape(1, D), rep, summ, tile]
    if qkvw is not None:
        qWl, kWl, vWl, qbl, kbl, vbl = qkvw
        in_specs += [full((D, D))] * 3 + [vec] * 3
        args += [qWl, kWl, vWl, qbl.reshape(1, D), kbl.reshape(1, D),
                 vbl.reshape(1, D)]
        hspec = pl.BlockSpec((H, BS, DH), lambda i: (0, i, 0))
        haspec = pl.BlockSpec((H, BS, 2 * DH), lambda i: (0, i, 0))
        out_specs = [blk, blk, hspec, hspec, haspec]
        out_shape = [jax.ShapeDtypeStruct((S, D), jnp.float32),
                     jax.ShapeDtypeStruct((S, D), jnp.float32),
                     jax.ShapeDtypeStruct((H, S, DH), jnp.bfloat16),
                     jax.ShapeDtypeStruct((H, S, DH), jnp.bfloat16),
                     jax.ShapeDtypeStruct((H, S, 2 * DH), jnp.bfloat16)]
    else:
        out_specs = [blk]
        out_shape = [jax.ShapeDtypeStruct((S, D), jnp.bfloat16)]
    return pl.pallas_call(
        functools.partial(_ffn_body, qkvw is not None),
        grid=(S // BS,),
        in_specs=in_specs,
        out_specs=out_specs,
        out_shape=out_shape,
    )(*args)


def _logits_body(h_ref, te_ref, o_ref):
    o_ref[...] = lax.dot_general(h_ref[...], te_ref[...].astype(jnp.bfloat16),
                                 (((1,), (1,)), ((), ())),
                                 preferred_element_type=jnp.float32)


def _logits(hfin, token_emb):
    v = token_emb.shape[0]
    return pl.pallas_call(
        _logits_body,
        grid=(v // VB, S // BS),
        in_specs=[pl.BlockSpec((BS, D), lambda vb, s: (s, 0)),
                  pl.BlockSpec((VB, D), lambda vb, s: (vb, 0))],
        out_specs=pl.BlockSpec((BS, VB), lambda vb, s: (s, vb)),
        out_shape=jax.ShapeDtypeStruct((S, v), jnp.float32),
    )(hfin, token_emb)


def kernel(input_ids, token_emb, pos_emb, qW, qb, kW, kb, vW, vb, sW, sb,
           recipe, WdW, Wdb, n1g, n1b, n2g, n2b, ng, nb,
           basis_A, basis_B, basis_emb):
    ids = input_ids.reshape(S).astype(jnp.int32)
    g = _embed_gather(token_emb, ids)

    Af = basis_A.transpose(1, 0, 2).reshape(D, NB * R)
    Bf = basis_B.reshape(NB * R, DFF)
    rep = jnp.kron(jnp.eye(NB, dtype=jnp.float32),
                   jnp.ones((1, R), jnp.float32))
    summ = jnp.tile(jnp.eye(R, dtype=jnp.float32), (NB, 1))
    tile = jnp.tile(jnp.eye(R, dtype=jnp.float32), (1, NB))

    Af = Af.astype(jnp.bfloat16)
    Bf = Bf.astype(jnp.bfloat16)
    WdW16 = WdW.astype(jnp.bfloat16)
    qW16 = qW.astype(jnp.bfloat16)
    kW16 = kW.astype(jnp.bfloat16)
    vW16 = vW.astype(jnp.bfloat16)
    x, h1, q, k, vaug = _embed_finish(g, pos_emb, n1g[0], n1b[0],
                                      qW16[0], kW16[0], vW16[0],
                                      qb[0], kb[0], vb[0])

    for l in range(L):
        ctx = _attn(q, k, vaug)
        if l < L - 1:
            x, h1, q, k, vaug = _ffn(
                x, h1, ctx, sW[l, :D], sW[l, D:], sb[l], recipe[l],
                basis_emb, Af, Bf, WdW16[l], Wdb[l], n2g[l], n2b[l],
                n1g[l + 1], n1b[l + 1], rep, summ, tile,
                qkvw=(qW16[l + 1], kW16[l + 1], vW16[l + 1],
                      qb[l + 1], kb[l + 1], vb[l + 1]))
        else:
            (hfin16,) = _ffn(
                x, h1, ctx, sW[l, :D], sW[l, D:], sb[l], recipe[l],
                basis_emb, Af, Bf, WdW16[l], Wdb[l], n2g[l], n2b[l],
                ng, nb, rep, summ, tile)

    logits = _logits(hfin16, token_emb)
    return logits.reshape(1, S, token_emb.shape[0])


# speedup vs baseline: 1.8958x; 1.0018x over previous
import functools

import jax
import jax.numpy as jnp
from jax import lax
from jax.experimental import pallas as pl
from jax.experimental.pallas import tpu as pltpu
from jax.experimental.pallas import tpu_sc as plsc

S, D, H, DH = 2048, 768, 12, 64
NB, R, DFF, NN, K = 32, 64, 1024, 64, 8
L = 4
BS = 256
ASB = 512
VB = 2048
EPS = 1e-5


def _lnorm(x, g, b):
    mu = jnp.mean(x, axis=-1, keepdims=True)
    var = jnp.mean((x - mu) ** 2, axis=-1, keepdims=True)
    return (x - mu) * lax.rsqrt(var + EPS) * g + b


def _embed_gather(table, ids):
    info = plsc.get_sparse_core_info()
    nw = info.num_cores * info.num_subcores
    n = ids.shape[0]
    bpw = n // nw
    d = table.shape[1]
    mesh = plsc.VectorSubcoreMesh(core_axis_name="c", subcore_axis_name="s")

    @functools.partial(
        pl.kernel, mesh=mesh,
        out_type=jax.ShapeDtypeStruct((n, d), table.dtype),
        scratch_types=[
            pltpu.VMEM((bpw,), jnp.int32),
            pltpu.VMEM((bpw, d), table.dtype),
            pltpu.SemaphoreType.DMA,
        ],
    )
    def gather_k(table_hbm, idx_hbm, out_hbm, idx_v, rows_v, sem):
        wid = lax.axis_index("s") * info.num_cores + lax.axis_index("c")
        base = wid * bpw
        pltpu.sync_copy(idx_hbm.at[pl.ds(base, bpw)], idx_v)
        pltpu.async_copy(table_hbm.at[idx_v], rows_v, sem).wait()
        pltpu.sync_copy(rows_v, out_hbm.at[pl.ds(base, bpw)])

    return gather_k(table, ids)


def _qkv_emit(h1b, qW_ref, kW_ref, vW_ref, qb_ref, kb_ref, vb_ref,
              q_ref, k_ref, vaug_ref):
    q = (jnp.dot(h1b, qW_ref[...], preferred_element_type=jnp.float32)
         + qb_ref[...]).astype(jnp.bfloat16)
    k = (jnp.dot(h1b, kW_ref[...], preferred_element_type=jnp.float32)
         + kb_ref[...]).astype(jnp.bfloat16)
    v = (jnp.dot(h1b, vW_ref[...], preferred_element_type=jnp.float32)
         + vb_ref[...]).astype(jnp.bfloat16)
    ones = jnp.ones((h1b.shape[0], DH), jnp.bfloat16)
    for h in range(H):
        q_ref[h] = q[:, h * DH:(h + 1) * DH]
        k_ref[h] = k[:, h * DH:(h + 1) * DH]
        vaug_ref[h] = jnp.concatenate([v[:, h * DH:(h + 1) * DH], ones],
                                      axis=-1)


def _embed_finish_body(g_ref, pos_ref, g1_ref, b1_ref,
                       qW_ref, kW_ref, vW_ref, qb_ref, kb_ref, vb_ref,
                       x_ref, q_ref, k_ref, vaug_ref):
    x = g_ref[...] + pos_ref[...]
    x_ref[...] = x
    h1 = _lnorm(x, g1_ref[...], b1_ref[...])
    _qkv_emit(h1.astype(jnp.bfloat16), qW_ref, kW_ref, vW_ref,
              qb_ref, kb_ref, vb_ref, q_ref, k_ref, vaug_ref)


def _embed_finish(g, pos, g1, b1, qWl, kWl, vWl, qbl, kbl, vbl):
    blk = pl.BlockSpec((BS, D), lambda i: (i, 0))
    vec = pl.BlockSpec((1, D), lambda i: (0, 0))
    wfull = pl.BlockSpec((D, D), lambda i: (0, 0))
    hspec = pl.BlockSpec((H, BS, DH), lambda i: (0, i, 0))
    haspec = pl.BlockSpec((H, BS, 2 * DH), lambda i: (0, i, 0))
    return pl.pallas_call(
        _embed_finish_body,
        grid=(S // BS,),
        in_specs=[blk, blk, vec, vec, wfull, wfull, wfull, vec, vec, vec],
        out_specs=[pl.BlockSpec((BS, D), lambda i: (i, 0)),
                   hspec, hspec, haspec],
        out_shape=[jax.ShapeDtypeStruct((S, D), jnp.float32),
                   jax.ShapeDtypeStruct((H, S, DH), jnp.bfloat16),
                   jax.ShapeDtypeStruct((H, S, DH), jnp.bfloat16),
                   jax.ShapeDtypeStruct((H, S, 2 * DH), jnp.bfloat16)],
    )(g, pos, g1.reshape(1, D), b1.reshape(1, D), qWl, kWl, vWl,
      qbl.reshape(1, D), kbl.reshape(1, D), vbl.reshape(1, D))


def _attn_body(q_ref, k_ref, vaug_ref, o_ref):
    k = k_ref[0]
    vaug = vaug_ref[0]
    for c in range(S // ASB):
        rows = pl.ds(c * ASB, ASB)
        scores = lax.dot_general(q_ref[0, rows, :], k,
                                 (((1,), (1,)), ((), ())),
                                 preferred_element_type=jnp.float32)
        scores = scores * (1.0 / (DH ** 0.5))
        m = jnp.max(scores, axis=-1, keepdims=True)
        p16 = jnp.exp(scores - m).astype(jnp.bfloat16)
        ctxd = jnp.dot(p16, vaug, preferred_element_type=jnp.float32)
        ctx = ctxd[:, :DH] / ctxd[:, DH:DH + 1]
        o_ref[0, rows, :] = ctx


def _attn(q, k, vaug):
    hspec = pl.BlockSpec((1, S, DH), lambda h: (h, 0, 0))
    haspec = pl.BlockSpec((1, S, 2 * DH), lambda h: (h, 0, 0))
    return pl.pallas_call(
        _attn_body,
        grid=(H,),
        in_specs=[hspec, hspec, haspec],
        out_specs=hspec,
        out_shape=jax.ShapeDtypeStruct((H, S, DH), jnp.float32),
    )(q, k, vaug)


def _ffn_body(with_qkv, x_ref, ctx_ref, sW1_ref, sW2_ref, sb_ref,
              rec_ref, bemb_ref, Af_ref, Bf_ref, WdW_ref, Wdb_ref,
              g1_ref, b1_ref, n2g_ref, n2b_ref, gn_ref, bn_ref,
              rep_ref, summ_ref, tile_ref, *rest):
    if with_qkv:
        (qW_ref, kW_ref, vW_ref, qb_ref, kb_ref, vb_ref,
         xo_ref, q_ref, k_ref, vaug_ref) = rest
    else:
        (h1o16_ref,) = rest
    x = x_ref[...]
    h1 = _lnorm(x, g1_ref[...], b1_ref[...])
    ctx = jnp.concatenate([ctx_ref[h] for h in range(H)], axis=-1)
    query = (jnp.dot(h1, sW1_ref[...], preferred_element_type=jnp.float32)
             + jnp.dot(ctx, sW2_ref[...], preferred_element_type=jnp.float32)
             + sb_ref[...])

    rec = rec_ref[...]
    er = jnp.exp(rec - jnp.max(rec, axis=-1, keepdims=True))
    srec = er / jnp.sum(er, axis=-1, keepdims=True)
    nemb = jnp.dot(srec, bemb_ref[...], preferred_element_type=jnp.float32)
    scores = lax.dot_general(query, nemb, (((1,), (1,)), ((), ())),
                             preferred_element_type=jnp.float32)

    iota = lax.broadcasted_iota(jnp.int32, (BS, NN), 1)
    work = scores
    selmask = jnp.zeros((BS, NN), jnp.bool_)
    for _ in range(K):
        cm = jnp.max(work, axis=-1, keepdims=True)
        cand = jnp.where(work == cm, iota, NN)
        first = jnp.min(cand, axis=-1, keepdims=True)
        onehot = iota == first
        selmask = jnp.logical_or(selmask, onehot)
        work = jnp.where(onehot, -jnp.inf, work)
    gmax = jnp.max(scores, axis=-1, keepdims=True)
    wnum = jnp.where(selmask, jnp.exp(scores - gmax), 0.0)
    wd = wnum / jnp.sum(wnum, axis=-1, keepdims=True)
    tr = jnp.dot(wd, srec, preferred_element_type=jnp.float32)

    h2 = _lnorm(x, n2g_ref[...], n2b_ref[...])
    u = jnp.dot(h2.astype(jnp.bfloat16), Af_ref[...],
                preferred_element_type=jnp.float32)
    t = jnp.dot(tr, rep_ref[...], preferred_element_type=jnp.float32)
    hh = jnp.dot(u * t, summ_ref[...], preferred_element_type=jnp.float32)
    hrep = jnp.dot(hh, tile_ref[...], preferred_element_type=jnp.float32)
    ff = jnp.dot((t * hrep).astype(jnp.bfloat16), Bf_ref[...],
                 preferred_element_type=jnp.float32)
    ff = ff * 0.5 * (1.0 + lax.erf(ff * (2.0 ** -0.5)))
    y = (jnp.dot(ff.astype(jnp.bfloat16), WdW_ref[...],
                 preferred_element_type=jnp.float32) + Wdb_ref[...])
    xn = x + y
    h1o = _lnorm(xn, gn_ref[...], bn_ref[...])
    if with_qkv:
        xo_ref[...] = xn
        _qkv_emit(h1o.astype(jnp.bfloat16), qW_ref, kW_ref, vW_ref,
                  qb_ref, kb_ref, vb_ref, q_ref, k_ref, vaug_ref)
    else:
        h1o16_ref[...] = h1o.astype(jnp.bfloat16)


def _ffn(x, ctx, sW1, sW2, sb, rec, bemb, Af, Bf, WdWl, Wdbl,
         g1, b1, n2gl, n2bl, gn, bn, rep, summ, tile, qkvw=None):
    blk = pl.BlockSpec((BS, D), lambda i: (i, 0))
    vec = pl.BlockSpec((1, D), lambda i: (0, 0))
    full = lambda shape: pl.BlockSpec(shape, lambda i: (0,) * len(shape))
    in_specs = [blk,
                pl.BlockSpec((H, BS, DH), lambda i: (0, i, 0)),
                full((D, D)), full((D, D)), vec,
                full((NN, NB)), full((NB, D)),
                full((D, NB * R)), full((NB * R, DFF)),
                full((DFF, D)), vec, vec, vec, vec, vec, vec, vec,
                full((NB, NB * R)), full((NB * R, R)), full((R, NB * R))]
    args = [x, ctx, sW1, sW2, sb.reshape(1, D), rec, bemb, Af, Bf, WdWl,
            Wdbl.reshape(1, D), g1.reshape(1, D), b1.reshape(1, D),
            n2gl.reshape(1, D), n2bl.reshape(1, D),
            gn.reshape(1, D), bn.reshape(1, D), rep, summ, tile]
    if qkvw is not None:
        qWl, kWl, vWl, qbl, kbl, vbl = qkvw
        in_specs += [full((D, D))] * 3 + [vec] * 3
        args += [qWl, kWl, vWl, qbl.reshape(1, D), kbl.reshape(1, D),
                 vbl.reshape(1, D)]
        hspec = pl.BlockSpec((H, BS, DH), lambda i: (0, i, 0))
        haspec = pl.BlockSpec((H, BS, 2 * DH), lambda i: (0, i, 0))
        out_specs = [blk, hspec, hspec, haspec]
        out_shape = [jax.ShapeDtypeStruct((S, D), jnp.float32),
                     jax.ShapeDtypeStruct((H, S, DH), jnp.bfloat16),
                     jax.ShapeDtypeStruct((H, S, DH), jnp.bfloat16),
                     jax.ShapeDtypeStruct((H, S, 2 * DH), jnp.bfloat16)]
    else:
        out_specs = [blk]
        out_shape = [jax.ShapeDtypeStruct((S, D), jnp.bfloat16)]
    return pl.pallas_call(
        functools.partial(_ffn_body, qkvw is not None),
        grid=(S // BS,),
        in_specs=in_specs,
        out_specs=out_specs,
        out_shape=out_shape,
    )(*args)


def _logits_body(h_ref, te_ref, o_ref):
    o_ref[...] = lax.dot_general(h_ref[...], te_ref[...].astype(jnp.bfloat16),
                                 (((1,), (1,)), ((), ())),
                                 preferred_element_type=jnp.float32)


def _logits(hfin, token_emb):
    v = token_emb.shape[0]
    return pl.pallas_call(
        _logits_body,
        grid=(v // VB, S // BS),
        in_specs=[pl.BlockSpec((BS, D), lambda vb, s: (s, 0)),
                  pl.BlockSpec((VB, D), lambda vb, s: (vb, 0))],
        out_specs=pl.BlockSpec((BS, VB), lambda vb, s: (s, vb)),
        out_shape=jax.ShapeDtypeStruct((S, v), jnp.float32),
    )(hfin, token_emb)


def kernel(input_ids, token_emb, pos_emb, qW, qb, kW, kb, vW, vb, sW, sb,
           recipe, WdW, Wdb, n1g, n1b, n2g, n2b, ng, nb,
           basis_A, basis_B, basis_emb):
    ids = input_ids.reshape(S).astype(jnp.int32)
    g = _embed_gather(token_emb, ids)

    Af = basis_A.transpose(1, 0, 2).reshape(D, NB * R)
    Bf = basis_B.reshape(NB * R, DFF)
    rep = jnp.kron(jnp.eye(NB, dtype=jnp.float32),
                   jnp.ones((1, R), jnp.float32))
    summ = jnp.tile(jnp.eye(R, dtype=jnp.float32), (NB, 1))
    tile = jnp.tile(jnp.eye(R, dtype=jnp.float32), (1, NB))

    Af = Af.astype(jnp.bfloat16)
    Bf = Bf.astype(jnp.bfloat16)
    WdW16 = WdW.astype(jnp.bfloat16)
    qW16 = qW.astype(jnp.bfloat16)
    kW16 = kW.astype(jnp.bfloat16)
    vW16 = vW.astype(jnp.bfloat16)
    x, q, k, vaug = _embed_finish(g, pos_emb, n1g[0], n1b[0],
                                  qW16[0], kW16[0], vW16[0],
                                  qb[0], kb[0], vb[0])

    for l in range(L):
        ctx = _attn(q, k, vaug)
        if l < L - 1:
            x, q, k, vaug = _ffn(
                x, ctx, sW[l, :D], sW[l, D:], sb[l], recipe[l],
                basis_emb, Af, Bf, WdW16[l], Wdb[l], n1g[l], n1b[l],
                n2g[l], n2b[l], n1g[l + 1], n1b[l + 1], rep, summ, tile,
                qkvw=(qW16[l + 1], kW16[l + 1], vW16[l + 1],
                      qb[l + 1], kb[l + 1], vb[l + 1]))
        else:
            (hfin16,) = _ffn(
                x, ctx, sW[l, :D], sW[l, D:], sb[l], recipe[l],
                basis_emb, Af, Bf, WdW16[l], Wdb[l], n1g[l], n1b[l],
                n2g[l], n2b[l], ng, nb, rep, summ, tile)

    logits = _logits(hfin16, token_emb)
    return logits.reshape(1, S, token_emb.shape[0])


# speedup vs baseline: 2.0593x; 1.0863x over previous
import functools

import jax
import jax.numpy as jnp
from jax import lax
from jax.experimental import pallas as pl
from jax.experimental.pallas import tpu as pltpu
from jax.experimental.pallas import tpu_sc as plsc

S, D, H, DH = 2048, 768, 12, 64
NB, R, DFF, NN, K = 32, 64, 1024, 64, 8
L = 4
BS = 512
ASB = 256
VB = 2048
EPS = 1e-5


def _lnorm(x, g, b):
    mu = jnp.mean(x, axis=-1, keepdims=True)
    var = jnp.mean((x - mu) ** 2, axis=-1, keepdims=True)
    return (x - mu) * lax.rsqrt(var + EPS) * g + b


def _embed_gather(table, ids):
    info = plsc.get_sparse_core_info()
    nw = info.num_cores * info.num_subcores
    n = ids.shape[0]
    bpw = n // nw
    d = table.shape[1]
    mesh = plsc.VectorSubcoreMesh(core_axis_name="c", subcore_axis_name="s")

    @functools.partial(
        pl.kernel, mesh=mesh,
        out_type=jax.ShapeDtypeStruct((n, d), table.dtype),
        scratch_types=[
            pltpu.VMEM((bpw,), jnp.int32),
            pltpu.VMEM((bpw, d), table.dtype),
            pltpu.SemaphoreType.DMA,
        ],
    )
    def gather_k(table_hbm, idx_hbm, out_hbm, idx_v, rows_v, sem):
        wid = lax.axis_index("s") * info.num_cores + lax.axis_index("c")
        base = wid * bpw
        pltpu.sync_copy(idx_hbm.at[pl.ds(base, bpw)], idx_v)
        pltpu.async_copy(table_hbm.at[idx_v], rows_v, sem).wait()
        pltpu.sync_copy(rows_v, out_hbm.at[pl.ds(base, bpw)])

    return gather_k(table, ids)


def _qkv_emit(h1b, qW_ref, kW_ref, vW_ref, qb_ref, kb_ref, vb_ref,
              q_ref, k_ref, vaug_ref):
    q = (jnp.dot(h1b, qW_ref[...], preferred_element_type=jnp.float32)
         + qb_ref[...]).astype(jnp.bfloat16)
    k = (jnp.dot(h1b, kW_ref[...], preferred_element_type=jnp.float32)
         + kb_ref[...]).astype(jnp.bfloat16)
    v = (jnp.dot(h1b, vW_ref[...], preferred_element_type=jnp.float32)
         + vb_ref[...]).astype(jnp.bfloat16)
    ones = jnp.ones((h1b.shape[0], DH), jnp.bfloat16)
    for h in range(H):
        q_ref[h] = q[:, h * DH:(h + 1) * DH]
        k_ref[h] = k[:, h * DH:(h + 1) * DH]
        vaug_ref[h] = jnp.concatenate([v[:, h * DH:(h + 1) * DH], ones],
                                      axis=-1)


def _embed_finish_body(g_ref, pos_ref, g1_ref, b1_ref,
                       qW_ref, kW_ref, vW_ref, qb_ref, kb_ref, vb_ref,
                       x_ref, q_ref, k_ref, vaug_ref):
    x = g_ref[...] + pos_ref[...]
    x_ref[...] = x
    h1 = _lnorm(x, g1_ref[...], b1_ref[...])
    _qkv_emit(h1.astype(jnp.bfloat16), qW_ref, kW_ref, vW_ref,
              qb_ref, kb_ref, vb_ref, q_ref, k_ref, vaug_ref)


def _embed_finish(g, pos, g1, b1, qWl, kWl, vWl, qbl, kbl, vbl):
    blk = pl.BlockSpec((BS, D), lambda i: (i, 0))
    vec = pl.BlockSpec((1, D), lambda i: (0, 0))
    wfull = pl.BlockSpec((D, D), lambda i: (0, 0))
    hspec = pl.BlockSpec((H, BS, DH), lambda i: (0, i, 0))
    haspec = pl.BlockSpec((H, BS, 2 * DH), lambda i: (0, i, 0))
    return pl.pallas_call(
        _embed_finish_body,
        grid=(S // BS,),
        in_specs=[blk, blk, vec, vec, wfull, wfull, wfull, vec, vec, vec],
        out_specs=[pl.BlockSpec((BS, D), lambda i: (i, 0)),
                   hspec, hspec, haspec],
        out_shape=[jax.ShapeDtypeStruct((S, D), jnp.float32),
                   jax.ShapeDtypeStruct((H, S, DH), jnp.bfloat16),
                   jax.ShapeDtypeStruct((H, S, DH), jnp.bfloat16),
                   jax.ShapeDtypeStruct((H, S, 2 * DH), jnp.bfloat16)],
    )(g, pos, g1.reshape(1, D), b1.reshape(1, D), qWl, kWl, vWl,
      qbl.reshape(1, D), kbl.reshape(1, D), vbl.reshape(1, D))


def _attn_body(q_ref, k_ref, vaug_ref, o_ref):
    k = k_ref[0]
    vaug = vaug_ref[0]
    for c in range(S // ASB):
        rows = pl.ds(c * ASB, ASB)
        scores = lax.dot_general(q_ref[0, rows, :], k,
                                 (((1,), (1,)), ((), ())),
                                 preferred_element_type=jnp.float32)
        m = jnp.max(scores, axis=-1, keepdims=True)
        p16 = jnp.exp2((scores - m).astype(jnp.bfloat16))
        ctxd = jnp.dot(p16, vaug, preferred_element_type=jnp.float32)
        ctx = ctxd[:, :DH] / ctxd[:, DH:DH + 1]
        o_ref[0, rows, :] = ctx.astype(jnp.bfloat16)


def _attn(q, k, vaug):
    hspec = pl.BlockSpec((1, S, DH), lambda h: (h, 0, 0))
    haspec = pl.BlockSpec((1, S, 2 * DH), lambda h: (h, 0, 0))
    return pl.pallas_call(
        _attn_body,
        grid=(H,),
        in_specs=[hspec, hspec, haspec],
        out_specs=hspec,
        out_shape=jax.ShapeDtypeStruct((H, S, DH), jnp.bfloat16),
    )(q, k, vaug)


def _ffn_body(with_qkv, x_ref, ctx_ref, sW1_ref, sW2_ref, sb_ref,
              rec_ref, bemb_ref, Af_ref, Bf_ref, WdW_ref, Wdb_ref,
              g1_ref, b1_ref, n2g_ref, n2b_ref, gn_ref, bn_ref,
              rep_ref, summ_ref, tile_ref, *rest):
    if with_qkv:
        (qW_ref, kW_ref, vW_ref, qb_ref, kb_ref, vb_ref,
         xo_ref, q_ref, k_ref, vaug_ref) = rest
    else:
        (h1o16_ref,) = rest
    x = x_ref[...]
    h1 = _lnorm(x, g1_ref[...], b1_ref[...])
    ctx = jnp.concatenate([ctx_ref[h] for h in range(H)], axis=-1)
    query = (jnp.dot(h1.astype(jnp.bfloat16), sW1_ref[...],
                     preferred_element_type=jnp.float32)
             + jnp.dot(ctx, sW2_ref[...], preferred_element_type=jnp.float32)
             + sb_ref[...])

    rec = rec_ref[...]
    er = jnp.exp(rec - jnp.max(rec, axis=-1, keepdims=True))
    srec = er / jnp.sum(er, axis=-1, keepdims=True)
    nemb = jnp.dot(srec, bemb_ref[...], preferred_element_type=jnp.float32)
    scores = lax.dot_general(query, nemb, (((1,), (1,)), ((), ())),
                             preferred_element_type=jnp.float32)

    work = scores
    selmask = jnp.zeros((BS, NN), jnp.bool_)
    for _ in range(K):
        cm = jnp.max(work, axis=-1, keepdims=True)
        onehot = work == cm
        selmask = jnp.logical_or(selmask, onehot)
        work = jnp.where(onehot, -jnp.inf, work)
    gmax = jnp.max(scores, axis=-1, keepdims=True)
    wnum = jnp.where(selmask, jnp.exp(scores - gmax), 0.0)
    wd = wnum / jnp.sum(wnum, axis=-1, keepdims=True)
    tr = jnp.dot(wd, srec, preferred_element_type=jnp.float32)

    h2 = _lnorm(x, n2g_ref[...], n2b_ref[...])
    u = jnp.dot(h2.astype(jnp.bfloat16), Af_ref[...],
                preferred_element_type=jnp.float32)
    t = jnp.dot(tr, rep_ref[...], preferred_element_type=jnp.float32)
    hh = jnp.dot(u * t, summ_ref[...], preferred_element_type=jnp.float32)
    hrep = jnp.dot(hh, tile_ref[...], preferred_element_type=jnp.float32)
    ff = jnp.dot((t * hrep).astype(jnp.bfloat16), Bf_ref[...],
                 preferred_element_type=jnp.float32)
    ff = ff * 0.5 * (1.0 + lax.erf(ff * (2.0 ** -0.5)))
    y = (jnp.dot(ff.astype(jnp.bfloat16), WdW_ref[...],
                 preferred_element_type=jnp.float32) + Wdb_ref[...])
    xn = x + y
    h1o = _lnorm(xn, gn_ref[...], bn_ref[...])
    if with_qkv:
        xo_ref[...] = xn
        _qkv_emit(h1o.astype(jnp.bfloat16), qW_ref, kW_ref, vW_ref,
                  qb_ref, kb_ref, vb_ref, q_ref, k_ref, vaug_ref)
    else:
        h1o16_ref[...] = h1o.astype(jnp.bfloat16)


def _ffn(x, ctx, sW1, sW2, sb, rec, bemb, Af, Bf, WdWl, Wdbl,
         g1, b1, n2gl, n2bl, gn, bn, rep, summ, tile, qkvw=None):
    blk = pl.BlockSpec((BS, D), lambda i: (i, 0))
    vec = pl.BlockSpec((1, D), lambda i: (0, 0))
    full = lambda shape: pl.BlockSpec(shape, lambda i: (0,) * len(shape))
    in_specs = [blk,
                pl.BlockSpec((H, BS, DH), lambda i: (0, i, 0)),
                full((D, D)), full((D, D)), vec,
                full((NN, NB)), full((NB, D)),
                full((D, NB * R)), full((NB * R, DFF)),
                full((DFF, D)), vec, vec, vec, vec, vec, vec, vec,
                full((NB, NB * R)), full((NB * R, R)), full((R, NB * R))]
    args = [x, ctx, sW1.astype(jnp.bfloat16), sW2.astype(jnp.bfloat16),
            sb.reshape(1, D), rec, bemb, Af, Bf, WdWl,
            Wdbl.reshape(1, D), g1.reshape(1, D), b1.reshape(1, D),
            n2gl.reshape(1, D), n2bl.reshape(1, D),
            gn.reshape(1, D), bn.reshape(1, D), rep, summ, tile]
    if qkvw is not None:
        qWl, kWl, vWl, qbl, kbl, vbl = qkvw
        in_specs += [full((D, D))] * 3 + [vec] * 3
        args += [qWl, kWl, vWl, qbl.reshape(1, D), kbl.reshape(1, D),
                 vbl.reshape(1, D)]
        hspec = pl.BlockSpec((H, BS, DH), lambda i: (0, i, 0))
        haspec = pl.BlockSpec((H, BS, 2 * DH), lambda i: (0, i, 0))
        out_specs = [blk, hspec, hspec, haspec]
        out_shape = [jax.ShapeDtypeStruct((S, D), jnp.float32),
                     jax.ShapeDtypeStruct((H, S, DH), jnp.bfloat16),
                     jax.ShapeDtypeStruct((H, S, DH), jnp.bfloat16),
                     jax.ShapeDtypeStruct((H, S, 2 * DH), jnp.bfloat16)]
    else:
        out_specs = [blk]
        out_shape = [jax.ShapeDtypeStruct((S, D), jnp.bfloat16)]
    return pl.pallas_call(
        functools.partial(_ffn_body, qkvw is not None),
        grid=(S // BS,),
        in_specs=in_specs,
        out_specs=out_specs,
        out_shape=out_shape,
    )(*args)


def _logits_body(h_ref, te_ref, o_ref):
    o_ref[...] = lax.dot_general(h_ref[...], te_ref[...].astype(jnp.bfloat16),
                                 (((1,), (1,)), ((), ())),
                                 preferred_element_type=jnp.float32)


def _logits(hfin, token_emb):
    v = token_emb.shape[0]
    return pl.pallas_call(
        _logits_body,
        grid=(v // VB, S // BS),
        in_specs=[pl.BlockSpec((BS, D), lambda vb, s: (s, 0)),
                  pl.BlockSpec((VB, D), lambda vb, s: (vb, 0))],
        out_specs=pl.BlockSpec((BS, VB), lambda vb, s: (s, vb)),
        out_shape=jax.ShapeDtypeStruct((S, v), jnp.float32),
    )(hfin, token_emb)


def kernel(input_ids, token_emb, pos_emb, qW, qb, kW, kb, vW, vb, sW, sb,
           recipe, WdW, Wdb, n1g, n1b, n2g, n2b, ng, nb,
           basis_A, basis_B, basis_emb):
    ids = input_ids.reshape(S).astype(jnp.int32)
    g = _embed_gather(token_emb, ids)

    Af = basis_A.transpose(1, 0, 2).reshape(D, NB * R)
    Bf = basis_B.reshape(NB * R, DFF)
    rep = jnp.kron(jnp.eye(NB, dtype=jnp.float32),
                   jnp.ones((1, R), jnp.float32))
    summ = jnp.tile(jnp.eye(R, dtype=jnp.float32), (NB, 1))
    tile = jnp.tile(jnp.eye(R, dtype=jnp.float32), (1, NB))

    Af = Af.astype(jnp.bfloat16)
    Bf = Bf.astype(jnp.bfloat16)
    WdW16 = WdW.astype(jnp.bfloat16)
    scale = 1.4426950408889634 / (DH ** 0.5)
    qW16 = (qW * scale).astype(jnp.bfloat16)
    kW16 = kW.astype(jnp.bfloat16)
    vW16 = vW.astype(jnp.bfloat16)
    qb = qb * scale
    x, q, k, vaug = _embed_finish(g, pos_emb, n1g[0], n1b[0],
                                  qW16[0], kW16[0], vW16[0],
                                  qb[0], kb[0], vb[0])

    for l in range(L):
        ctx = _attn(q, k, vaug)
        if l < L - 1:
            x, q, k, vaug = _ffn(
                x, ctx, sW[l, :D], sW[l, D:], sb[l], recipe[l],
                basis_emb, Af, Bf, WdW16[l], Wdb[l], n1g[l], n1b[l],
                n2g[l], n2b[l], n1g[l + 1], n1b[l + 1], rep, summ, tile,
                qkvw=(qW16[l + 1], kW16[l + 1], vW16[l + 1],
                      qb[l + 1], kb[l + 1], vb[l + 1]))
        else:
            (hfin16,) = _ffn(
                x, ctx, sW[l, :D], sW[l, D:], sb[l], recipe[l],
                basis_emb, Af, Bf, WdW16[l], Wdb[l], n1g[l], n1b[l],
                n2g[l], n2b[l], ng, nb, rep, summ, tile)

    logits = _logits(hfin16, token_emb)
    return logits.reshape(1, S, token_emb.shape[0])


# speedup vs baseline: 2.0794x; 1.0098x over previous
import functools

import jax
import jax.numpy as jnp
from jax import lax
from jax.experimental import pallas as pl
from jax.experimental.pallas import tpu as pltpu
from jax.experimental.pallas import tpu_sc as plsc

S, D, H, DH = 2048, 768, 12, 64
NB, R, DFF, NN, K = 32, 64, 1024, 64, 8
L = 4
BS = 512
ASB = 256
VB = 2048
EPS = 1e-5


def _lnorm(x, g, b):
    mu = jnp.mean(x, axis=-1, keepdims=True)
    var = jnp.mean((x - mu) ** 2, axis=-1, keepdims=True)
    return (x - mu) * lax.rsqrt(var + EPS) * g + b


def _embed_gather(table, ids):
    info = plsc.get_sparse_core_info()
    nw = info.num_cores * info.num_subcores
    n = ids.shape[0]
    bpw = n // nw
    d = table.shape[1]
    mesh = plsc.VectorSubcoreMesh(core_axis_name="c", subcore_axis_name="s")

    @functools.partial(
        pl.kernel, mesh=mesh,
        out_type=jax.ShapeDtypeStruct((n, d), table.dtype),
        scratch_types=[
            pltpu.VMEM((bpw,), jnp.int32),
            pltpu.VMEM((bpw, d), table.dtype),
            pltpu.SemaphoreType.DMA,
        ],
    )
    def gather_k(table_hbm, idx_hbm, out_hbm, idx_v, rows_v, sem):
        wid = lax.axis_index("s") * info.num_cores + lax.axis_index("c")
        base = wid * bpw
        pltpu.sync_copy(idx_hbm.at[pl.ds(base, bpw)], idx_v)
        pltpu.async_copy(table_hbm.at[idx_v], rows_v, sem).wait()
        pltpu.sync_copy(rows_v, out_hbm.at[pl.ds(base, bpw)])

    return gather_k(table, ids)


def _qkv_emit(h1b, qW_ref, kW_ref, vW_ref, qb_ref, kb_ref, vb_ref,
              q_ref, k_ref, vaug_ref):
    q = (jnp.dot(h1b, qW_ref[...], preferred_element_type=jnp.float32)
         + qb_ref[...]).astype(jnp.bfloat16)
    k = (jnp.dot(h1b, kW_ref[...], preferred_element_type=jnp.float32)
         + kb_ref[...]).astype(jnp.bfloat16)
    v = (jnp.dot(h1b, vW_ref[...], preferred_element_type=jnp.float32)
         + vb_ref[...]).astype(jnp.bfloat16)
    ones = jnp.ones((h1b.shape[0], DH), jnp.bfloat16)
    for h in range(H):
        q_ref[h] = q[:, h * DH:(h + 1) * DH]
        k_ref[h] = k[:, h * DH:(h + 1) * DH]
        vaug_ref[h] = jnp.concatenate([v[:, h * DH:(h + 1) * DH], ones],
                                      axis=-1)


def _embed_finish_body(g_ref, pos_ref, g1_ref, b1_ref,
                       qW_ref, kW_ref, vW_ref, qb_ref, kb_ref, vb_ref,
                       x_ref, q_ref, k_ref, vaug_ref):
    x = g_ref[...] + pos_ref[...]
    x_ref[...] = x
    h1 = _lnorm(x, g1_ref[...], b1_ref[...])
    _qkv_emit(h1.astype(jnp.bfloat16), qW_ref, kW_ref, vW_ref,
              qb_ref, kb_ref, vb_ref, q_ref, k_ref, vaug_ref)


def _embed_finish(g, pos, g1, b1, qWl, kWl, vWl, qbl, kbl, vbl):
    blk = pl.BlockSpec((BS, D), lambda i: (i, 0))
    vec = pl.BlockSpec((1, D), lambda i: (0, 0))
    wfull = pl.BlockSpec((D, D), lambda i: (0, 0))
    hspec = pl.BlockSpec((H, BS, DH), lambda i: (0, i, 0))
    haspec = pl.BlockSpec((H, BS, 2 * DH), lambda i: (0, i, 0))
    return pl.pallas_call(
        _embed_finish_body,
        grid=(S // BS,),
        in_specs=[blk, blk, vec, vec, wfull, wfull, wfull, vec, vec, vec],
        out_specs=[pl.BlockSpec((BS, D), lambda i: (i, 0)),
                   hspec, hspec, haspec],
        out_shape=[jax.ShapeDtypeStruct((S, D), jnp.float32),
                   jax.ShapeDtypeStruct((H, S, DH), jnp.bfloat16),
                   jax.ShapeDtypeStruct((H, S, DH), jnp.bfloat16),
                   jax.ShapeDtypeStruct((H, S, 2 * DH), jnp.bfloat16)],
    )(g, pos, g1.reshape(1, D), b1.reshape(1, D), qWl, kWl, vWl,
      qbl.reshape(1, D), kbl.reshape(1, D), vbl.reshape(1, D))


HP = 2


def _attn_body(q_ref, k_ref, vaug_ref, o_ref):
    for hh in range(HP):
        k = k_ref[hh]
        vaug = vaug_ref[hh]
        for c in range(S // ASB):
            rows = pl.ds(c * ASB, ASB)
            scores = lax.dot_general(q_ref[hh, rows, :], k,
                                     (((1,), (1,)), ((), ())),
                                     preferred_element_type=jnp.float32)
            m = jnp.max(scores, axis=-1, keepdims=True)
            p16 = jnp.exp2((scores - m).astype(jnp.bfloat16))
            ctxd = jnp.dot(p16, vaug, preferred_element_type=jnp.float32)
            ctx = ctxd[:, :DH] / ctxd[:, DH:DH + 1]
            o_ref[hh, rows, :] = ctx.astype(jnp.bfloat16)


def _attn(q, k, vaug):
    hspec = pl.BlockSpec((HP, S, DH), lambda h: (h, 0, 0))
    haspec = pl.BlockSpec((HP, S, 2 * DH), lambda h: (h, 0, 0))
    return pl.pallas_call(
        _attn_body,
        grid=(H // HP,),
        in_specs=[hspec, hspec, haspec],
        out_specs=hspec,
        out_shape=jax.ShapeDtypeStruct((H, S, DH), jnp.bfloat16),
    )(q, k, vaug)


def _ffn_body(with_qkv, x_ref, ctx_ref, sW1_ref, sW2_ref, sb_ref,
              rec_ref, bemb_ref, Af_ref, Bf_ref, WdW_ref, Wdb_ref,
              g1_ref, b1_ref, n2g_ref, n2b_ref, gn_ref, bn_ref,
              rep_ref, summ_ref, tile_ref, *rest):
    if with_qkv:
        (qW_ref, kW_ref, vW_ref, qb_ref, kb_ref, vb_ref,
         xo_ref, q_ref, k_ref, vaug_ref) = rest
    else:
        (h1o16_ref,) = rest
    x = x_ref[...]
    h1 = _lnorm(x, g1_ref[...], b1_ref[...])
    ctx = jnp.concatenate([ctx_ref[h] for h in range(H)], axis=-1)
    query = (jnp.dot(h1.astype(jnp.bfloat16), sW1_ref[...],
                     preferred_element_type=jnp.float32)
             + jnp.dot(ctx, sW2_ref[...], preferred_element_type=jnp.float32)
             + sb_ref[...])

    rec = rec_ref[...]
    er = jnp.exp(rec - jnp.max(rec, axis=-1, keepdims=True))
    srec = er / jnp.sum(er, axis=-1, keepdims=True)
    nemb = jnp.dot(srec, bemb_ref[...], preferred_element_type=jnp.float32)
    scores = lax.dot_general(query, nemb, (((1,), (1,)), ((), ())),
                             preferred_element_type=jnp.float32)

    work = scores
    selmask = jnp.zeros((BS, NN), jnp.bool_)
    for _ in range(K):
        cm = jnp.max(work, axis=-1, keepdims=True)
        onehot = work == cm
        selmask = jnp.logical_or(selmask, onehot)
        work = jnp.where(onehot, -jnp.inf, work)
    gmax = jnp.max(scores, axis=-1, keepdims=True)
    wnum = jnp.where(selmask, jnp.exp(scores - gmax), 0.0)
    wd = wnum / jnp.sum(wnum, axis=-1, keepdims=True)
    tr = jnp.dot(wd, srec, preferred_element_type=jnp.float32)

    h2 = _lnorm(x, n2g_ref[...], n2b_ref[...])
    u = jnp.dot(h2.astype(jnp.bfloat16), Af_ref[...],
                preferred_element_type=jnp.float32)
    t = jnp.dot(tr, rep_ref[...], preferred_element_type=jnp.float32)
    hh = jnp.dot(u * t, summ_ref[...], preferred_element_type=jnp.float32)
    hrep = jnp.dot(hh, tile_ref[...], preferred_element_type=jnp.float32)
    ff = jnp.dot((t * hrep).astype(jnp.bfloat16), Bf_ref[...],
                 preferred_element_type=jnp.float32)
    ff = ff * 0.5 * (1.0 + lax.erf(ff * (2.0 ** -0.5)))
    y = (jnp.dot(ff.astype(jnp.bfloat16), WdW_ref[...],
                 preferred_element_type=jnp.float32) + Wdb_ref[...])
    xn = x + y
    h1o = _lnorm(xn, gn_ref[...], bn_ref[...])
    if with_qkv:
        xo_ref[...] = xn
        _qkv_emit(h1o.astype(jnp.bfloat16), qW_ref, kW_ref, vW_ref,
                  qb_ref, kb_ref, vb_ref, q_ref, k_ref, vaug_ref)
    else:
        h1o16_ref[...] = h1o.astype(jnp.bfloat16)


def _ffn(x, ctx, sW1, sW2, sb, rec, bemb, Af, Bf, WdWl, Wdbl,
         g1, b1, n2gl, n2bl, gn, bn, rep, summ, tile, qkvw=None):
    blk = pl.BlockSpec((BS, D), lambda i: (i, 0))
    vec = pl.BlockSpec((1, D), lambda i: (0, 0))
    full = lambda shape: pl.BlockSpec(shape, lambda i: (0,) * len(shape))
    in_specs = [blk,
                pl.BlockSpec((H, BS, DH), lambda i: (0, i, 0)),
                full((D, D)), full((D, D)), vec,
                full((NN, NB)), full((NB, D)),
                full((D, NB * R)), full((NB * R, DFF)),
                full((DFF, D)), vec, vec, vec, vec, vec, vec, vec,
                full((NB, NB * R)), full((NB * R, R)), full((R, NB * R))]
    args = [x, ctx, sW1.astype(jnp.bfloat16), sW2.astype(jnp.bfloat16),
            sb.reshape(1, D), rec, bemb, Af, Bf, WdWl,
            Wdbl.reshape(1, D), g1.reshape(1, D), b1.reshape(1, D),
            n2gl.reshape(1, D), n2bl.reshape(1, D),
            gn.reshape(1, D), bn.reshape(1, D), rep, summ, tile]
    if qkvw is not None:
        qWl, kWl, vWl, qbl, kbl, vbl = qkvw
        in_specs += [full((D, D))] * 3 + [vec] * 3
        args += [qWl, kWl, vWl, qbl.reshape(1, D), kbl.reshape(1, D),
                 vbl.reshape(1, D)]
        hspec = pl.BlockSpec((H, BS, DH), lambda i: (0, i, 0))
        haspec = pl.BlockSpec((H, BS, 2 * DH), lambda i: (0, i, 0))
        out_specs = [blk, hspec, hspec, haspec]
        out_shape = [jax.ShapeDtypeStruct((S, D), jnp.float32),
                     jax.ShapeDtypeStruct((H, S, DH), jnp.bfloat16),
                     jax.ShapeDtypeStruct((H, S, DH), jnp.bfloat16),
                     jax.ShapeDtypeStruct((H, S, 2 * DH), jnp.bfloat16)]
    else:
        out_specs = [blk]
        out_shape = [jax.ShapeDtypeStruct((S, D), jnp.bfloat16)]
    return pl.pallas_call(
        functools.partial(_ffn_body, qkvw is not None),
        grid=(S // BS,),
        in_specs=in_specs,
        out_specs=out_specs,
        out_shape=out_shape,
    )(*args)


def _logits_body(h_ref, te_ref, o_ref):
    o_ref[...] = lax.dot_general(h_ref[...], te_ref[...].astype(jnp.bfloat16),
                                 (((1,), (1,)), ((), ())),
                                 preferred_element_type=jnp.float32)


def _logits(hfin, token_emb):
    v = token_emb.shape[0]
    return pl.pallas_call(
        _logits_body,
        grid=(v // VB, S // BS),
        in_specs=[pl.BlockSpec((BS, D), lambda vb, s: (s, 0)),
                  pl.BlockSpec((VB, D), lambda vb, s: (vb, 0))],
        out_specs=pl.BlockSpec((BS, VB), lambda vb, s: (s, vb)),
        out_shape=jax.ShapeDtypeStruct((S, v), jnp.float32),
    )(hfin, token_emb)


def kernel(input_ids, token_emb, pos_emb, qW, qb, kW, kb, vW, vb, sW, sb,
           recipe, WdW, Wdb, n1g, n1b, n2g, n2b, ng, nb,
           basis_A, basis_B, basis_emb):
    ids = input_ids.reshape(S).astype(jnp.int32)
    g = _embed_gather(token_emb, ids)

    Af = basis_A.transpose(1, 0, 2).reshape(D, NB * R)
    Bf = basis_B.reshape(NB * R, DFF)
    rep = jnp.kron(jnp.eye(NB, dtype=jnp.float32),
                   jnp.ones((1, R), jnp.float32))
    summ = jnp.tile(jnp.eye(R, dtype=jnp.float32), (NB, 1))
    tile = jnp.tile(jnp.eye(R, dtype=jnp.float32), (1, NB))

    Af = Af.astype(jnp.bfloat16)
    Bf = Bf.astype(jnp.bfloat16)
    WdW16 = WdW.astype(jnp.bfloat16)
    scale = 1.4426950408889634 / (DH ** 0.5)
    qW16 = (qW * scale).astype(jnp.bfloat16)
    kW16 = kW.astype(jnp.bfloat16)
    vW16 = vW.astype(jnp.bfloat16)
    qb = qb * scale
    x, q, k, vaug = _embed_finish(g, pos_emb, n1g[0], n1b[0],
                                  qW16[0], kW16[0], vW16[0],
                                  qb[0], kb[0], vb[0])

    for l in range(L):
        ctx = _attn(q, k, vaug)
        if l < L - 1:
            x, q, k, vaug = _ffn(
                x, ctx, sW[l, :D], sW[l, D:], sb[l], recipe[l],
                basis_emb, Af, Bf, WdW16[l], Wdb[l], n1g[l], n1b[l],
                n2g[l], n2b[l], n1g[l + 1], n1b[l + 1], rep, summ, tile,
                qkvw=(qW16[l + 1], kW16[l + 1], vW16[l + 1],
                      qb[l + 1], kb[l + 1], vb[l + 1]))
        else:
            (hfin16,) = _ffn(
                x, ctx, sW[l, :D], sW[l, D:], sb[l], recipe[l],
                basis_emb, Af, Bf, WdW16[l], Wdb[l], n1g[l], n1b[l],
                n2g[l], n2b[l], ng, nb, rep, summ, tile)

    logits = _logits(hfin16, token_emb)
    return logits.reshape(1, S, token_emb.shape[0])
